# baseline (speedup 1.0000x reference)
import functools

import jax
import jax.numpy as jnp
from jax import lax
from jax.experimental import pallas as pl
from jax.experimental.pallas import tpu as pltpu
from jax.experimental.pallas import tpu_sc as plsc

_N = 10000
_E = 320000
_G = 200
_F = 64

_NC = 2
_NS = 16
_NW = _NC * _NS
_CHUNK = 128
_A = 0
_B = 160
_TOT_CHUNKS = _NS * (_A + _B)
_E_PAD = _TOT_CHUNKS * _CHUNK
_N_ACC = 10112
_RPT = _N_ACC // _NS

@functools.lru_cache(maxsize=None)
def _mesh():
    return plsc.VectorSubcoreMesh(core_axis_name="c", subcore_axis_name="s",
                                  num_cores=_NC, num_subcores=_NS)


def _sc_degree(dst3):
    k = pl.kernel(
        _sc_degree_body,
        mesh=_mesh(),
        out_type=jax.ShapeDtypeStruct((_NC, _N_ACC, 16), jnp.float32),
        scratch_types=[
            pltpu.VMEM((max(_A, _B), _CHUNK), jnp.int32),
            pltpu.VMEM((_CHUNK, 16), jnp.float32),
            pltpu.VMEM((_RPT, 16), jnp.float32),
            pltpu.VMEM_SHARED((_N_ACC, 16), jnp.float32),
            pltpu.SemaphoreType.DMA,
        ],
        compiler_params=pltpu.CompilerParams(use_tc_tiling_on_sc=False),
    )
    return k(dst3)


def _sc_degree_body(dst_hbm, out_hbm, dsts_v, ones_v, zrows_v, acc_sh, sem):
    cid = lax.axis_index("c")
    sid = lax.axis_index("s")
    one16 = jnp.full((16,), 1.0, jnp.float32)
    zero16 = jnp.zeros((16,), jnp.float32)

    def fill(j, _):
        ones_v[j, :] = one16
        return 0

    lax.fori_loop(0, _CHUNK, fill, 0)

    def zfill(j, _):
        zrows_v[j, :] = zero16
        return 0

    lax.fori_loop(0, _RPT, zfill, 0)

    row_base = sid * _RPT
    pltpu.sync_copy(zrows_v, acc_sh.at[pl.ds(row_base, _RPT)])

    def run(nchunks, start_row):
        if nchunks == 0:
            return
        pltpu.sync_copy(dst_hbm.at[pl.ds(start_row, nchunks)],
                        dsts_v.at[pl.ds(0, nchunks)])
        plsc.subcore_barrier()

        def body(j, _):
            for b in range(8):
                pltpu.async_copy(ones_v, acc_sh.at[dsts_v.at[j * 8 + b]], sem, add=True)
            for b in range(8):
                pltpu.make_async_copy(ones_v, acc_sh.at[dsts_v.at[j * 8 + b]], sem).wait()
            return 0

        lax.fori_loop(0, nchunks // 8, body, 0)

    @pl.when(cid == 0)
    def _():
        run(_A, sid * _A)

    @pl.when(cid == 1)
    def _():
        run(_B, _NS * _A + sid * _B)

    plsc.subcore_barrier()
    pltpu.sync_copy(acc_sh.at[pl.ds(row_base, _RPT)],
                    out_hbm.at[cid, pl.ds(row_base, _RPT)])


_NBUF = 4


def _sc_scatter(g, src3, dst3):
    k = pl.kernel(
        _sc_scatter_body,
        mesh=_mesh(),
        out_type=jax.ShapeDtypeStruct((_NC, _N_ACC, _F), jnp.float32),
        scratch_types=[
            pltpu.VMEM((max(_A, _B), _CHUNK), jnp.int32),
            pltpu.VMEM((max(_A, _B), _CHUNK), jnp.int32),
            pltpu.VMEM((_NBUF, _CHUNK, _F), jnp.float32),
            pltpu.VMEM_SHARED((_N_ACC, _F), jnp.float32),
            pltpu.SemaphoreType.DMA,
            pltpu.SemaphoreType.DMA,
            pltpu.SemaphoreType.DMA,
            pltpu.SemaphoreType.DMA,
            pltpu.SemaphoreType.DMA,
        ],
        compiler_params=pltpu.CompilerParams(use_tc_tiling_on_sc=False),
    )
    return k(g, src3, dst3)


def _sc_scatter_body(g_hbm, src_hbm, dst_hbm, out_hbm, srcs_v, dsts_v, rows_v,
                     acc_sh, gs0, gs1, gs2, gs3, ssem):
    cid = lax.axis_index("c")
    sid = lax.axis_index("s")
    gsems = [gs0, gs1, gs2, gs3]
    zero16 = jnp.zeros((16,), jnp.float32)

    def zfill(j, _):
        for gc in range(_F // 16):
            rows_v[0, j, pl.ds(gc * 16, 16)] = zero16
        return 0

    lax.fori_loop(0, _CHUNK, zfill, 0)

    row_base = sid * _RPT
    for z in range(4):
        pltpu.sync_copy(rows_v.at[0], acc_sh.at[pl.ds(row_base + z * _CHUNK, _CHUNK)])
    pltpu.sync_copy(rows_v.at[0, pl.ds(0, _RPT - 4 * _CHUNK)],
                    acc_sh.at[pl.ds(row_base + 4 * _CHUNK, _RPT - 4 * _CHUNK)])

    def gstart(c, b):
        pltpu.async_copy(g_hbm.at[srcs_v.at[c]], rows_v.at[b], gsems[b])

    def gwait(c, b):
        pltpu.make_async_copy(g_hbm.at[srcs_v.at[c]], rows_v.at[b], gsems[b]).wait()

    def sstep(c, b):
        pltpu.async_copy(rows_v.at[b], acc_sh.at[dsts_v.at[c]], ssem, add=True)
        pltpu.make_async_copy(rows_v.at[b], acc_sh.at[dsts_v.at[c]], ssem).wait()

    def run(nchunks, start_row):
        if nchunks == 0:
            return
        pltpu.sync_copy(src_hbm.at[pl.ds(start_row, nchunks)],
                        srcs_v.at[pl.ds(0, nchunks)])
        pltpu.sync_copy(dst_hbm.at[pl.ds(start_row, nchunks)],
                        dsts_v.at[pl.ds(0, nchunks)])
        plsc.subcore_barrier()
        for b in range(_NBUF):
            gstart(b, b)

        def body(j, _):
            for b in range(_NBUF):
                c = j * _NBUF + b
                gwait(c, b)
                sstep(c, b)
                gstart(c + _NBUF, b)
            return 0

        lax.fori_loop(0, nchunks // _NBUF - 1, body, 0)
        for b in range(_NBUF):
            c = nchunks - _NBUF + b
            gwait(c, b)
            sstep(c, b)

    @pl.when(cid == 0)
    def _():
        run(_A, sid * _A)

    @pl.when(cid == 1)
    def _():
        run(_B, _NS * _A + sid * _B)

    plsc.subcore_barrier()
    pltpu.sync_copy(acc_sh.at[pl.ds(row_base, _RPT)],
                    out_hbm.at[cid, pl.ds(row_base, _RPT)])


_BLK = 1000


def _tc_prep(x, degp, W1):
    def body(x_ref, degp_ref, W1_ref, g_ref, dis_ref):
        deg = 1.0 + degp_ref[0] + degp_ref[1]
        dis = lax.rsqrt(deg)
        dis_ref[...] = dis
        g_ref[...] = jnp.dot(x_ref[...], W1_ref[...],
                             preferred_element_type=jnp.float32) * dis

    return pl.pallas_call(
        body,
        grid=(_N // _BLK,),
        in_specs=[
            pl.BlockSpec((_BLK, 128), lambda i: (i, 0)),
            pl.BlockSpec((2, _BLK, 1), lambda i: (0, i, 0)),
            pl.BlockSpec((128, _F), lambda i: (0, 0)),
        ],
        out_specs=[
            pl.BlockSpec((_BLK, _F), lambda i: (i, 0)),
            pl.BlockSpec((_BLK, 1), lambda i: (i, 0)),
        ],
        out_shape=[
            jax.ShapeDtypeStruct((_N, _F), jnp.float32),
            jax.ShapeDtypeStruct((_N, 1), jnp.float32),
        ],
    )(x, degp, W1)


def _tc_mid(accp, g, dis, b, W):
    def body(accp_ref, g_ref, dis_ref, b_ref, W_ref, gn_ref):
        dis = dis_ref[...]
        h = jnp.maximum(dis * (accp_ref[0] + accp_ref[1] + g_ref[...]) + b_ref[...], 0.0)
        gn_ref[...] = jnp.dot(h, W_ref[...], preferred_element_type=jnp.float32) * dis

    return pl.pallas_call(
        body,
        grid=(_N // _BLK,),
        in_specs=[
            pl.BlockSpec((2, _BLK, _F), lambda i: (0, i, 0)),
            pl.BlockSpec((_BLK, _F), lambda i: (i, 0)),
            pl.BlockSpec((_BLK, 1), lambda i: (i, 0)),
            pl.BlockSpec((1, _F), lambda i: (0, 0)),
            pl.BlockSpec((_F, _F), lambda i: (0, 0)),
        ],
        out_specs=pl.BlockSpec((_BLK, _F), lambda i: (i, 0)),
        out_shape=jax.ShapeDtypeStruct((_N, _F), jnp.float32),
    )(accp, g, dis, b, W)


_FBLK = 400
_NFB = _N // _FBLK


def _tc_final(accp, g, dis, b, bcol, brow, bnext, mol, fWa, fWb, fWc, fb, f2W, f2b):
    def body(accp_ref, g_ref, dis_ref, b_ref, bcol_ref, brow_ref, bnext_ref,
             mol_ref, fWa_ref, fWb_ref, fWc_ref, fb_ref, f2W_ref, f2b_ref,
             out_ref, gmax_s, gsum_s, cnt_s, cm_s, cb_s):
        i = pl.program_id(0)

        @pl.when(i == 0)
        def _():
            gmax_s[...] = jnp.zeros((_G, _F), jnp.float32)
            gsum_s[...] = jnp.zeros((_G, _F), jnp.float32)
            cnt_s[...] = jnp.zeros((_G, 1), jnp.float32)
            cm_s[...] = jnp.zeros((1, _F), jnp.float32)
            cb_s[...] = jnp.full((1, 1), -1, jnp.int32)

        h = jnp.maximum(
            dis_ref[...] * (accp_ref[0] + accp_ref[1] + g_ref[...]) + b_ref[...], 0.0)
        bcol = bcol_ref[...]
        m = h
        for k in range(9):
            sh = 1 << k
            m_sh = jnp.concatenate(
                [jnp.zeros((sh, _F), jnp.float32), m[:_FBLK - sh]], axis=0)
            b_sh = jnp.concatenate(
                [jnp.full((sh, 1), -1, jnp.int32), bcol[:_FBLK - sh]], axis=0)
            m = jnp.where(bcol == b_sh, jnp.maximum(m, m_sh), m)
        m = jnp.where(bcol == cb_s[...], jnp.maximum(m, cm_s[...]), m)
        cm_s[...] = m[_FBLK - 1:_FBLK]
        cb_s[...] = bcol[_FBLK - 1:_FBLK]

        brow = brow_ref[0]
        gids = lax.broadcasted_iota(jnp.int32, (_G, _FBLK), 0)
        member = (gids == brow)
        soh = member.astype(jnp.float32)
        eoh = (member & (brow != bnext_ref[0])).astype(jnp.float32)
        gmax_s[...] += jnp.dot(eoh, m, preferred_element_type=jnp.float32)
        gsum_s[...] += jnp.dot(soh, h, preferred_element_type=jnp.float32)
        cnt_s[...] += jnp.sum(soh, axis=1, keepdims=True)

        @pl.when(i == _NFB - 1)
        def _():
            gmean = gsum_s[...] / jnp.maximum(cnt_s[...], 1.0)
            cc = (jnp.dot(gmax_s[...], fWa_ref[...], preferred_element_type=jnp.float32)
                  + jnp.dot(gmean, fWb_ref[...], preferred_element_type=jnp.float32)
                  + jnp.dot(mol_ref[...], fWc_ref[...], preferred_element_type=jnp.float32)
                  + fb_ref[...])
            cc = jnp.maximum(cc, 0.0)
            out_ref[...] = (jnp.dot(cc, f2W_ref[...], preferred_element_type=jnp.float32)
                            + f2b_ref[...])

    return pl.pallas_call(
        body,
        grid=(_NFB,),
        in_specs=[
            pl.BlockSpec((2, _FBLK, _F), lambda i: (0, i, 0)),
            pl.BlockSpec((_FBLK, _F), lambda i: (i, 0)),
            pl.BlockSpec((_FBLK, 1), lambda i: (i, 0)),
            pl.BlockSpec((1, _F), lambda i: (0, 0)),
            pl.BlockSpec((_FBLK, 1), lambda i: (i, 0)),
            pl.BlockSpec((1, 1, _FBLK), lambda i: (i, 0, 0)),
            pl.BlockSpec((1, 1, _FBLK), lambda i: (i, 0, 0)),
            pl.BlockSpec((_G, 16), lambda i: (0, 0)),
            pl.BlockSpec((_F, _F), lambda i: (0, 0)),
            pl.BlockSpec((_F, _F), lambda i: (0, 0)),
            pl.BlockSpec((16, _F), lambda i: (0, 0)),
            pl.BlockSpec((1, _F), lambda i: (0, 0)),
            pl.BlockSpec((_F, 1), lambda i: (0, 0)),
            pl.BlockSpec((1, 1), lambda i: (0, 0)),
        ],
        out_specs=pl.BlockSpec((_G, 1), lambda i: (0, 0)),
        out_shape=jax.ShapeDtypeStruct((_G, 1), jnp.float32),
        scratch_shapes=[
            pltpu.VMEM((_G, _F), jnp.float32),
            pltpu.VMEM((_G, _F), jnp.float32),
            pltpu.VMEM((_G, 1), jnp.float32),
            pltpu.VMEM((1, _F), jnp.float32),
            pltpu.VMEM((1, 1), jnp.int32),
        ],
    )(accp, g, dis, b, bcol, brow, bnext, mol, fWa, fWb, fWc, fb, f2W, f2b)


def kernel(x, edge_index, batch, mol_features, W1, b1, W2, b2, W3, b3,
           fc1_W, fc1_b, fc2_W, fc2_b):
    src = edge_index[0].astype(jnp.int32)
    dst = edge_index[1].astype(jnp.int32)
    pad = _E_PAD - _E
    src_p = jnp.concatenate([src, jnp.zeros((pad,), jnp.int32)]).reshape(_TOT_CHUNKS, _CHUNK)
    dst_p = jnp.concatenate([dst, jnp.full((pad,), _N, jnp.int32)]).reshape(_TOT_CHUNKS, _CHUNK)
    bat = batch.astype(jnp.int32)
    bcol = bat.reshape(_N, 1)
    brow = bat.reshape(_NFB, 1, _FBLK)
    bnext = jnp.concatenate([bat[1:], jnp.full((1,), -1, jnp.int32)]).reshape(_NFB, 1, _FBLK)

    degp = _sc_degree(dst_p)[:, :_N, 0:1]
    g1, dis = _tc_prep(x, degp, W1)
    acc1 = _sc_scatter(g1, src_p, dst_p)[:, :_N, :]
    g2 = _tc_mid(acc1, g1, dis, b1.reshape(1, _F), W2)
    acc2 = _sc_scatter(g2, src_p, dst_p)[:, :_N, :]
    g3 = _tc_mid(acc2, g2, dis, b2.reshape(1, _F), W3)
    acc3 = _sc_scatter(g3, src_p, dst_p)[:, :_N, :]
    out = _tc_final(acc3, g3, dis, b3.reshape(1, _F), bcol, brow, bnext,
                    mol_features, fc1_W[:_F], fc1_W[_F:2 * _F], fc1_W[2 * _F:],
                    fc1_b.reshape(1, _F), fc2_W, fc2_b.reshape(1, 1))
    return out[:, 0]


# speedup vs baseline: 1.1282x; 1.1282x over previous
import functools

import jax
import jax.numpy as jnp
from jax import lax
from jax.experimental import pallas as pl
from jax.experimental.pallas import tpu as pltpu
from jax.experimental.pallas import tpu_sc as plsc

_N = 10000
_E = 320000
_G = 200
_F = 64

_NC = 2
_NS = 16
_NW = _NC * _NS
_CHUNK = 128
_A = 104
_B = 56
_TOT_CHUNKS = _NS * (_A + _B)
_E_PAD = _TOT_CHUNKS * _CHUNK
_N_ACC = 10112
_RPT = _N_ACC // _NS

@functools.lru_cache(maxsize=None)
def _mesh():
    return plsc.VectorSubcoreMesh(core_axis_name="c", subcore_axis_name="s",
                                  num_cores=_NC, num_subcores=_NS)


def _sc_degree(dst3):
    k = pl.kernel(
        _sc_degree_body,
        mesh=_mesh(),
        out_type=jax.ShapeDtypeStruct((_NC, _N_ACC, 16), jnp.float32),
        scratch_types=[
            pltpu.VMEM((max(_A, _B), _CHUNK), jnp.int32),
            pltpu.VMEM((_CHUNK, 16), jnp.float32),
            pltpu.VMEM((_RPT, 16), jnp.float32),
            pltpu.VMEM_SHARED((_N_ACC, 16), jnp.float32),
            pltpu.SemaphoreType.DMA,
        ],
        compiler_params=pltpu.CompilerParams(use_tc_tiling_on_sc=False),
    )
    return k(dst3)


def _sc_degree_body(dst_hbm, out_hbm, dsts_v, ones_v, zrows_v, acc_sh, sem):
    cid = lax.axis_index("c")
    sid = lax.axis_index("s")
    one16 = jnp.full((16,), 1.0, jnp.float32)
    zero16 = jnp.zeros((16,), jnp.float32)

    def fill(j, _):
        ones_v[j, :] = one16
        return 0

    lax.fori_loop(0, _CHUNK, fill, 0)

    def zfill(j, _):
        zrows_v[j, :] = zero16
        return 0

    lax.fori_loop(0, _RPT, zfill, 0)

    row_base = sid * _RPT
    pltpu.sync_copy(zrows_v, acc_sh.at[pl.ds(row_base, _RPT)])

    def run(nchunks, start_row):
        if nchunks == 0:
            return
        pltpu.sync_copy(dst_hbm.at[pl.ds(start_row, nchunks)],
                        dsts_v.at[pl.ds(0, nchunks)])
        plsc.subcore_barrier()

        def body(j, _):
            for b in range(8):
                pltpu.async_copy(ones_v, acc_sh.at[dsts_v.at[j * 8 + b]], sem, add=True)
            for b in range(8):
                pltpu.make_async_copy(ones_v, acc_sh.at[dsts_v.at[j * 8 + b]], sem).wait()
            return 0

        lax.fori_loop(0, nchunks // 8, body, 0)

    @pl.when(cid == 0)
    def _():
        run(_A, sid * _A)

    @pl.when(cid == 1)
    def _():
        run(_B, _NS * _A + sid * _B)

    plsc.subcore_barrier()
    pltpu.sync_copy(acc_sh.at[pl.ds(row_base, _RPT)],
                    out_hbm.at[cid, pl.ds(row_base, _RPT)])


_NBUF = 4


def _sc_scatter(g, src3, dst3):
    k = pl.kernel(
        _sc_scatter_body,
        mesh=_mesh(),
        out_type=jax.ShapeDtypeStruct((_NC, _N_ACC, _F), jnp.float32),
        scratch_types=[
            pltpu.VMEM((max(_A, _B), _CHUNK), jnp.int32),
            pltpu.VMEM((max(_A, _B), _CHUNK), jnp.int32),
            pltpu.VMEM((_NBUF, _CHUNK, _F), jnp.float32),
            pltpu.VMEM_SHARED((_N_ACC, _F), jnp.float32),
            pltpu.SemaphoreType.DMA,
            pltpu.SemaphoreType.DMA,
            pltpu.SemaphoreType.DMA,
            pltpu.SemaphoreType.DMA,
            pltpu.SemaphoreType.DMA,
        ],
        compiler_params=pltpu.CompilerParams(use_tc_tiling_on_sc=False),
    )
    return k(g, src3, dst3)


def _sc_scatter_body(g_hbm, src_hbm, dst_hbm, out_hbm, srcs_v, dsts_v, rows_v,
                     acc_sh, gs0, gs1, gs2, gs3, ssem):
    cid = lax.axis_index("c")
    sid = lax.axis_index("s")
    gsems = [gs0, gs1, gs2, gs3]
    zero16 = jnp.zeros((16,), jnp.float32)

    def zfill(j, _):
        for gc in range(_F // 16):
            rows_v[0, j, pl.ds(gc * 16, 16)] = zero16
        return 0

    lax.fori_loop(0, _CHUNK, zfill, 0)

    row_base = sid * _RPT
    for z in range(4):
        pltpu.sync_copy(rows_v.at[0], acc_sh.at[pl.ds(row_base + z * _CHUNK, _CHUNK)])
    pltpu.sync_copy(rows_v.at[0, pl.ds(0, _RPT - 4 * _CHUNK)],
                    acc_sh.at[pl.ds(row_base + 4 * _CHUNK, _RPT - 4 * _CHUNK)])

    def gstart(c, b):
        pltpu.async_copy(g_hbm.at[srcs_v.at[c]], rows_v.at[b], gsems[b])

    def gwait(c, b):
        pltpu.make_async_copy(g_hbm.at[srcs_v.at[c]], rows_v.at[b], gsems[b]).wait()

    def sstep(c, b):
        pltpu.async_copy(rows_v.at[b], acc_sh.at[dsts_v.at[c]], ssem, add=True)
        pltpu.make_async_copy(rows_v.at[b], acc_sh.at[dsts_v.at[c]], ssem).wait()

    def run(nchunks, start_row):
        if nchunks == 0:
            return
        pltpu.sync_copy(src_hbm.at[pl.ds(start_row, nchunks)],
                        srcs_v.at[pl.ds(0, nchunks)])
        pltpu.sync_copy(dst_hbm.at[pl.ds(start_row, nchunks)],
                        dsts_v.at[pl.ds(0, nchunks)])
        plsc.subcore_barrier()
        for b in range(_NBUF):
            gstart(b, b)

        def body(j, _):
            for b in range(_NBUF):
                c = j * _NBUF + b
                gwait(c, b)
                sstep(c, b)
                gstart(c + _NBUF, b)
            return 0

        lax.fori_loop(0, nchunks // _NBUF - 1, body, 0)
        for b in range(_NBUF):
            c = nchunks - _NBUF + b
            gwait(c, b)
            sstep(c, b)

    @pl.when(cid == 0)
    def _():
        run(_A, sid * _A)

    @pl.when(cid == 1)
    def _():
        run(_B, _NS * _A + sid * _B)

    plsc.subcore_barrier()
    pltpu.sync_copy(acc_sh.at[pl.ds(row_base, _RPT)],
                    out_hbm.at[cid, pl.ds(row_base, _RPT)])


_BLK = 1000


def _tc_prep(x, degp, W1):
    def body(x_ref, degp_ref, W1_ref, g_ref, dis_ref):
        deg = 1.0 + degp_ref[0] + degp_ref[1]
        dis = lax.rsqrt(deg)
        dis_ref[...] = dis
        g_ref[...] = jnp.dot(x_ref[...], W1_ref[...],
                             preferred_element_type=jnp.float32) * dis

    return pl.pallas_call(
        body,
        grid=(_N // _BLK,),
        in_specs=[
            pl.BlockSpec((_BLK, 128), lambda i: (i, 0)),
            pl.BlockSpec((2, _BLK, 1), lambda i: (0, i, 0)),
            pl.BlockSpec((128, _F), lambda i: (0, 0)),
        ],
        out_specs=[
            pl.BlockSpec((_BLK, _F), lambda i: (i, 0)),
            pl.BlockSpec((_BLK, 1), lambda i: (i, 0)),
        ],
        out_shape=[
            jax.ShapeDtypeStruct((_N, _F), jnp.float32),
            jax.ShapeDtypeStruct((_N, 1), jnp.float32),
        ],
    )(x, degp, W1)


def _tc_mid(accp, g, dis, b, W):
    def body(accp_ref, g_ref, dis_ref, b_ref, W_ref, gn_ref):
        dis = dis_ref[...]
        h = jnp.maximum(dis * (accp_ref[0] + accp_ref[1] + g_ref[...]) + b_ref[...], 0.0)
        gn_ref[...] = jnp.dot(h, W_ref[...], preferred_element_type=jnp.float32) * dis

    return pl.pallas_call(
        body,
        grid=(_N // _BLK,),
        in_specs=[
            pl.BlockSpec((2, _BLK, _F), lambda i: (0, i, 0)),
            pl.BlockSpec((_BLK, _F), lambda i: (i, 0)),
            pl.BlockSpec((_BLK, 1), lambda i: (i, 0)),
            pl.BlockSpec((1, _F), lambda i: (0, 0)),
            pl.BlockSpec((_F, _F), lambda i: (0, 0)),
        ],
        out_specs=pl.BlockSpec((_BLK, _F), lambda i: (i, 0)),
        out_shape=jax.ShapeDtypeStruct((_N, _F), jnp.float32),
    )(accp, g, dis, b, W)


_FBLK = 400
_NFB = _N // _FBLK


def _tc_final(accp, g, dis, b, bcol, brow, bnext, mol, fWa, fWb, fWc, fb, f2W, f2b):
    def body(accp_ref, g_ref, dis_ref, b_ref, bcol_ref, brow_ref, bnext_ref,
             mol_ref, fWa_ref, fWb_ref, fWc_ref, fb_ref, f2W_ref, f2b_ref,
             out_ref, gmax_s, gsum_s, cnt_s, cm_s, cb_s):
        i = pl.program_id(0)

        @pl.when(i == 0)
        def _():
            gmax_s[...] = jnp.zeros((_G, _F), jnp.float32)
            gsum_s[...] = jnp.zeros((_G, _F), jnp.float32)
            cnt_s[...] = jnp.zeros((_G, 1), jnp.float32)
            cm_s[...] = jnp.zeros((1, _F), jnp.float32)
            cb_s[...] = jnp.full((1, 1), -1, jnp.int32)

        h = jnp.maximum(
            dis_ref[...] * (accp_ref[0] + accp_ref[1] + g_ref[...]) + b_ref[...], 0.0)
        bcol = bcol_ref[...]
        m = h
        for k in range(9):
            sh = 1 << k
            m_sh = jnp.concatenate(
                [jnp.zeros((sh, _F), jnp.float32), m[:_FBLK - sh]], axis=0)
            b_sh = jnp.concatenate(
                [jnp.full((sh, 1), -1, jnp.int32), bcol[:_FBLK - sh]], axis=0)
            m = jnp.where(bcol == b_sh, jnp.maximum(m, m_sh), m)
        m = jnp.where(bcol == cb_s[...], jnp.maximum(m, cm_s[...]), m)
        cm_s[...] = m[_FBLK - 1:_FBLK]
        cb_s[...] = bcol[_FBLK - 1:_FBLK]

        brow = brow_ref[0]
        gids = lax.broadcasted_iota(jnp.int32, (_G, _FBLK), 0)
        member = (gids == brow)
        soh = member.astype(jnp.float32)
        eoh = (member & (brow != bnext_ref[0])).astype(jnp.float32)
        gmax_s[...] += jnp.dot(eoh, m, preferred_element_type=jnp.float32)
        gsum_s[...] += jnp.dot(soh, h, preferred_element_type=jnp.float32)
        cnt_s[...] += jnp.sum(soh, axis=1, keepdims=True)

        @pl.when(i == _NFB - 1)
        def _():
            gmean = gsum_s[...] / jnp.maximum(cnt_s[...], 1.0)
            cc = (jnp.dot(gmax_s[...], fWa_ref[...], preferred_element_type=jnp.float32)
                  + jnp.dot(gmean, fWb_ref[...], preferred_element_type=jnp.float32)
                  + jnp.dot(mol_ref[...], fWc_ref[...], preferred_element_type=jnp.float32)
                  + fb_ref[...])
            cc = jnp.maximum(cc, 0.0)
            out_ref[...] = (jnp.dot(cc, f2W_ref[...], preferred_element_type=jnp.float32)
                            + f2b_ref[...])

    return pl.pallas_call(
        body,
        grid=(_NFB,),
        in_specs=[
            pl.BlockSpec((2, _FBLK, _F), lambda i: (0, i, 0)),
            pl.BlockSpec((_FBLK, _F), lambda i: (i, 0)),
            pl.BlockSpec((_FBLK, 1), lambda i: (i, 0)),
            pl.BlockSpec((1, _F), lambda i: (0, 0)),
            pl.BlockSpec((_FBLK, 1), lambda i: (i, 0)),
            pl.BlockSpec((1, 1, _FBLK), lambda i: (i, 0, 0)),
            pl.BlockSpec((1, 1, _FBLK), lambda i: (i, 0, 0)),
            pl.BlockSpec((_G, 16), lambda i: (0, 0)),
            pl.BlockSpec((_F, _F), lambda i: (0, 0)),
            pl.BlockSpec((_F, _F), lambda i: (0, 0)),
            pl.BlockSpec((16, _F), lambda i: (0, 0)),
            pl.BlockSpec((1, _F), lambda i: (0, 0)),
            pl.BlockSpec((_F, 1), lambda i: (0, 0)),
            pl.BlockSpec((1, 1), lambda i: (0, 0)),
        ],
        out_specs=pl.BlockSpec((_G, 1), lambda i: (0, 0)),
        out_shape=jax.ShapeDtypeStruct((_G, 1), jnp.float32),
        scratch_shapes=[
            pltpu.VMEM((_G, _F), jnp.float32),
            pltpu.VMEM((_G, _F), jnp.float32),
            pltpu.VMEM((_G, 1), jnp.float32),
            pltpu.VMEM((1, _F), jnp.float32),
            pltpu.VMEM((1, 1), jnp.int32),
        ],
    )(accp, g, dis, b, bcol, brow, bnext, mol, fWa, fWb, fWc, fb, f2W, f2b)


def kernel(x, edge_index, batch, mol_features, W1, b1, W2, b2, W3, b3,
           fc1_W, fc1_b, fc2_W, fc2_b):
    src = edge_index[0].astype(jnp.int32)
    dst = edge_index[1].astype(jnp.int32)
    pad = _E_PAD - _E
    src_p = jnp.concatenate([src, jnp.zeros((pad,), jnp.int32)]).reshape(_TOT_CHUNKS, _CHUNK)
    dst_p = jnp.concatenate([dst, jnp.full((pad,), _N, jnp.int32)]).reshape(_TOT_CHUNKS, _CHUNK)
    bat = batch.astype(jnp.int32)
    bcol = bat.reshape(_N, 1)
    brow = bat.reshape(_NFB, 1, _FBLK)
    bnext = jnp.concatenate([bat[1:], jnp.full((1,), -1, jnp.int32)]).reshape(_NFB, 1, _FBLK)

    degp = _sc_degree(dst_p)[:, :_N, 0:1]
    g1, dis = _tc_prep(x, degp, W1)
    acc1 = _sc_scatter(g1, src_p, dst_p)[:, :_N, :]
    g2 = _tc_mid(acc1, g1, dis, b1.reshape(1, _F), W2)
    acc2 = _sc_scatter(g2, src_p, dst_p)[:, :_N, :]
    g3 = _tc_mid(acc2, g2, dis, b2.reshape(1, _F), W3)
    acc3 = _sc_scatter(g3, src_p, dst_p)[:, :_N, :]
    out = _tc_final(acc3, g3, dis, b3.reshape(1, _F), bcol, brow, bnext,
                    mol_features, fc1_W[:_F], fc1_W[_F:2 * _F], fc1_W[2 * _F:],
                    fc1_b.reshape(1, _F), fc2_W, fc2_b.reshape(1, 1))
    return out[:, 0]


# speedup vs baseline: 2.1557x; 1.9108x over previous
import functools

import jax
import jax.numpy as jnp
from jax import lax
from jax.experimental import pallas as pl
from jax.experimental.pallas import tpu as pltpu
from jax.experimental.pallas import tpu_sc as plsc

_N = 10000
_E = 320000
_G = 200
_F = 64

_NC = 2
_NS = 16
_NW = _NC * _NS
_CHUNK = 128
_A = 80
_B = 80
_TOT_CHUNKS = _NS * (_A + _B)
_E_PAD = _TOT_CHUNKS * _CHUNK
_N_ACC = 10112
_RPT = _N_ACC // _NS

@functools.lru_cache(maxsize=None)
def _mesh():
    return plsc.VectorSubcoreMesh(core_axis_name="c", subcore_axis_name="s",
                                  num_cores=_NC, num_subcores=_NS)


def _sc_degree(dst3):
    k = pl.kernel(
        _sc_degree_body,
        mesh=_mesh(),
        out_type=jax.ShapeDtypeStruct((_NC, _N_ACC, 16), jnp.float32),
        scratch_types=[
            pltpu.VMEM((max(_A, _B), _CHUNK), jnp.int32),
            pltpu.VMEM((_CHUNK, 16), jnp.float32),
            pltpu.VMEM((_RPT, 16), jnp.float32),
            pltpu.VMEM_SHARED((_N_ACC, 16), jnp.float32),
            pltpu.SemaphoreType.DMA,
        ],
        compiler_params=pltpu.CompilerParams(use_tc_tiling_on_sc=False),
    )
    return k(dst3)


def _sc_degree_body(dst_hbm, out_hbm, dsts_v, ones_v, zrows_v, acc_sh, sem):
    cid = lax.axis_index("c")
    sid = lax.axis_index("s")
    one16 = jnp.full((16,), 1.0, jnp.float32)
    zero16 = jnp.zeros((16,), jnp.float32)

    def fill(j, _):
        ones_v[j, :] = one16
        return 0

    lax.fori_loop(0, _CHUNK, fill, 0)

    def zfill(j, _):
        zrows_v[j, :] = zero16
        return 0

    lax.fori_loop(0, _RPT, zfill, 0)

    row_base = sid * _RPT
    pltpu.sync_copy(zrows_v, acc_sh.at[pl.ds(row_base, _RPT)])

    def run(nchunks, start_row):
        if nchunks == 0:
            return
        pltpu.sync_copy(dst_hbm.at[pl.ds(start_row, nchunks)],
                        dsts_v.at[pl.ds(0, nchunks)])
        plsc.subcore_barrier()

        def body(j, _):
            for b in range(8):
                pltpu.async_copy(ones_v, acc_sh.at[dsts_v.at[j * 8 + b]], sem, add=True)
            for b in range(8):
                pltpu.make_async_copy(ones_v, acc_sh.at[dsts_v.at[j * 8 + b]], sem).wait()
            return 0

        lax.fori_loop(0, nchunks // 8, body, 0)

    @pl.when(cid == 0)
    def _():
        run(_A, sid * _A)

    @pl.when(cid == 1)
    def _():
        run(_B, _NS * _A + sid * _B)

    plsc.subcore_barrier()
    pltpu.sync_copy(acc_sh.at[pl.ds(row_base, _RPT)],
                    out_hbm.at[cid, pl.ds(row_base, _RPT)])


_NBUF = 2


def _sc_scatter(g, src3, dst3):
    k = pl.kernel(
        _sc_scatter_body,
        mesh=_mesh(),
        out_type=jax.ShapeDtypeStruct((_NC, _N_ACC, _F), jnp.float32),
        scratch_types=[
            pltpu.VMEM((max(_A, _B), _CHUNK), jnp.int32),
            pltpu.VMEM((max(_A, _B), _CHUNK), jnp.int32),
            pltpu.VMEM((_NBUF, _CHUNK, _F), jnp.float32),
            pltpu.VMEM_SHARED((_N_ACC, _F), jnp.float32),
            pltpu.VMEM_SHARED((_N_ACC, _F), jnp.float32),
            pltpu.SemaphoreType.DMA,
            pltpu.SemaphoreType.DMA,
            pltpu.SemaphoreType.DMA,
            pltpu.SemaphoreType.DMA,
            pltpu.SemaphoreType.DMA,
        ],
        compiler_params=pltpu.CompilerParams(use_tc_tiling_on_sc=False),
    )
    return k(g, src3, dst3)


def _sc_scatter_body(g_hbm, src_hbm, dst_hbm, out_hbm, srcs_v, dsts_v, rows_v,
                     acc_sh, g_sh, gs0, gs1, gs2, gs3, ssem):
    cid = lax.axis_index("c")
    sid = lax.axis_index("s")
    gsems = [gs0, gs1, gs2, gs3]
    zero16 = jnp.zeros((16,), jnp.float32)

    def zfill(j, _):
        for gc in range(_F // 16):
            rows_v[0, j, pl.ds(gc * 16, 16)] = zero16
        return 0

    lax.fori_loop(0, _CHUNK, zfill, 0)

    row_base = sid * _RPT
    for z in range(4):
        pltpu.sync_copy(rows_v.at[0], acc_sh.at[pl.ds(row_base + z * _CHUNK, _CHUNK)])
    pltpu.sync_copy(rows_v.at[0, pl.ds(0, _RPT - 4 * _CHUNK)],
                    acc_sh.at[pl.ds(row_base + 4 * _CHUNK, _RPT - 4 * _CHUNK)])

    @pl.when(sid < _NS - 1)
    def _():
        pltpu.sync_copy(g_hbm.at[pl.ds(row_base, _RPT)],
                        g_sh.at[pl.ds(row_base, _RPT)])

    @pl.when(sid == _NS - 1)
    def _():
        pltpu.sync_copy(g_hbm.at[pl.ds((_NS - 1) * _RPT, _N - (_NS - 1) * _RPT)],
                        g_sh.at[pl.ds((_NS - 1) * _RPT, _N - (_NS - 1) * _RPT)])

    def gstart(c, b):
        pltpu.async_copy(g_sh.at[srcs_v.at[c]], rows_v.at[b], gsems[b])

    def gwait(c, b):
        pltpu.make_async_copy(g_sh.at[srcs_v.at[c]], rows_v.at[b], gsems[b]).wait()

    def sstep(c, b):
        pltpu.async_copy(rows_v.at[b], acc_sh.at[dsts_v.at[c]], ssem, add=True)
        pltpu.make_async_copy(rows_v.at[b], acc_sh.at[dsts_v.at[c]], ssem).wait()

    def run(nchunks, start_row):
        if nchunks == 0:
            return
        pltpu.sync_copy(src_hbm.at[pl.ds(start_row, nchunks)],
                        srcs_v.at[pl.ds(0, nchunks)])
        pltpu.sync_copy(dst_hbm.at[pl.ds(start_row, nchunks)],
                        dsts_v.at[pl.ds(0, nchunks)])
        plsc.subcore_barrier()
        for b in range(_NBUF):
            gstart(b, b)

        def body(j, _):
            for b in range(_NBUF):
                c = j * _NBUF + b
                gwait(c, b)
                sstep(c, b)
                gstart(c + _NBUF, b)
            return 0

        lax.fori_loop(0, nchunks // _NBUF - 1, body, 0)
        for b in range(_NBUF):
            c = nchunks - _NBUF + b
            gwait(c, b)
            sstep(c, b)

    @pl.when(cid == 0)
    def _():
        run(_A, sid * _A)

    @pl.when(cid == 1)
    def _():
        run(_B, _NS * _A + sid * _B)

    plsc.subcore_barrier()
    pltpu.sync_copy(acc_sh.at[pl.ds(row_base, _RPT)],
                    out_hbm.at[cid, pl.ds(row_base, _RPT)])


_BLK = 1000


def _tc_prep(x, degp, W1):
    def body(x_ref, degp_ref, W1_ref, g_ref, dis_ref):
        deg = 1.0 + degp_ref[0] + degp_ref[1]
        dis = lax.rsqrt(deg)
        dis_ref[...] = dis
        g_ref[...] = jnp.dot(x_ref[...], W1_ref[...],
                             preferred_element_type=jnp.float32) * dis

    return pl.pallas_call(
        body,
        grid=(_N // _BLK,),
        in_specs=[
            pl.BlockSpec((_BLK, 128), lambda i: (i, 0)),
            pl.BlockSpec((2, _BLK, 1), lambda i: (0, i, 0)),
            pl.BlockSpec((128, _F), lambda i: (0, 0)),
        ],
        out_specs=[
            pl.BlockSpec((_BLK, _F), lambda i: (i, 0)),
            pl.BlockSpec((_BLK, 1), lambda i: (i, 0)),
        ],
        out_shape=[
            jax.ShapeDtypeStruct((_N, _F), jnp.float32),
            jax.ShapeDtypeStruct((_N, 1), jnp.float32),
        ],
    )(x, degp, W1)


def _tc_mid(accp, g, dis, b, W):
    def body(accp_ref, g_ref, dis_ref, b_ref, W_ref, gn_ref):
        dis = dis_ref[...]
        h = jnp.maximum(dis * (accp_ref[0] + accp_ref[1] + g_ref[...]) + b_ref[...], 0.0)
        gn_ref[...] = jnp.dot(h, W_ref[...], preferred_element_type=jnp.float32) * dis

    return pl.pallas_call(
        body,
        grid=(_N // _BLK,),
        in_specs=[
            pl.BlockSpec((2, _BLK, _F), lambda i: (0, i, 0)),
            pl.BlockSpec((_BLK, _F), lambda i: (i, 0)),
            pl.BlockSpec((_BLK, 1), lambda i: (i, 0)),
            pl.BlockSpec((1, _F), lambda i: (0, 0)),
            pl.BlockSpec((_F, _F), lambda i: (0, 0)),
        ],
        out_specs=pl.BlockSpec((_BLK, _F), lambda i: (i, 0)),
        out_shape=jax.ShapeDtypeStruct((_N, _F), jnp.float32),
    )(accp, g, dis, b, W)


_FBLK = 400
_NFB = _N // _FBLK


def _tc_final(accp, g, dis, b, bcol, brow, bnext, mol, fWa, fWb, fWc, fb, f2W, f2b):
    def body(accp_ref, g_ref, dis_ref, b_ref, bcol_ref, brow_ref, bnext_ref,
             mol_ref, fWa_ref, fWb_ref, fWc_ref, fb_ref, f2W_ref, f2b_ref,
             out_ref, gmax_s, gsum_s, cnt_s, cm_s, cb_s):
        i = pl.program_id(0)

        @pl.when(i == 0)
        def _():
            gmax_s[...] = jnp.zeros((_G, _F), jnp.float32)
            gsum_s[...] = jnp.zeros((_G, _F), jnp.float32)
            cnt_s[...] = jnp.zeros((_G, 1), jnp.float32)
            cm_s[...] = jnp.zeros((1, _F), jnp.float32)
            cb_s[...] = jnp.full((1, 1), -1, jnp.int32)

        h = jnp.maximum(
            dis_ref[...] * (accp_ref[0] + accp_ref[1] + g_ref[...]) + b_ref[...], 0.0)
        bcol = bcol_ref[...]
        m = h
        for k in range(9):
            sh = 1 << k
            m_sh = jnp.concatenate(
                [jnp.zeros((sh, _F), jnp.float32), m[:_FBLK - sh]], axis=0)
            b_sh = jnp.concatenate(
                [jnp.full((sh, 1), -1, jnp.int32), bcol[:_FBLK - sh]], axis=0)
            m = jnp.where(bcol == b_sh, jnp.maximum(m, m_sh), m)
        m = jnp.where(bcol == cb_s[...], jnp.maximum(m, cm_s[...]), m)
        cm_s[...] = m[_FBLK - 1:_FBLK]
        cb_s[...] = bcol[_FBLK - 1:_FBLK]

        brow = brow_ref[0]
        gids = lax.broadcasted_iota(jnp.int32, (_G, _FBLK), 0)
        member = (gids == brow)
        soh = member.astype(jnp.float32)
        eoh = (member & (brow != bnext_ref[0])).astype(jnp.float32)
        gmax_s[...] += jnp.dot(eoh, m, preferred_element_type=jnp.float32)
        gsum_s[...] += jnp.dot(soh, h, preferred_element_type=jnp.float32)
        cnt_s[...] += jnp.sum(soh, axis=1, keepdims=True)

        @pl.when(i == _NFB - 1)
        def _():
            gmean = gsum_s[...] / jnp.maximum(cnt_s[...], 1.0)
            cc = (jnp.dot(gmax_s[...], fWa_ref[...], preferred_element_type=jnp.float32)
                  + jnp.dot(gmean, fWb_ref[...], preferred_element_type=jnp.float32)
                  + jnp.dot(mol_ref[...], fWc_ref[...], preferred_element_type=jnp.float32)
                  + fb_ref[...])
            cc = jnp.maximum(cc, 0.0)
            out_ref[...] = (jnp.dot(cc, f2W_ref[...], preferred_element_type=jnp.float32)
                            + f2b_ref[...])

    return pl.pallas_call(
        body,
        grid=(_NFB,),
        in_specs=[
            pl.BlockSpec((2, _FBLK, _F), lambda i: (0, i, 0)),
            pl.BlockSpec((_FBLK, _F), lambda i: (i, 0)),
            pl.BlockSpec((_FBLK, 1), lambda i: (i, 0)),
            pl.BlockSpec((1, _F), lambda i: (0, 0)),
            pl.BlockSpec((_FBLK, 1), lambda i: (i, 0)),
            pl.BlockSpec((1, 1, _FBLK), lambda i: (i, 0, 0)),
            pl.BlockSpec((1, 1, _FBLK), lambda i: (i, 0, 0)),
            pl.BlockSpec((_G, 16), lambda i: (0, 0)),
            pl.BlockSpec((_F, _F), lambda i: (0, 0)),
            pl.BlockSpec((_F, _F), lambda i: (0, 0)),
            pl.BlockSpec((16, _F), lambda i: (0, 0)),
            pl.BlockSpec((1, _F), lambda i: (0, 0)),
            pl.BlockSpec((_F, 1), lambda i: (0, 0)),
            pl.BlockSpec((1, 1), lambda i: (0, 0)),
        ],
        out_specs=pl.BlockSpec((_G, 1), lambda i: (0, 0)),
        out_shape=jax.ShapeDtypeStruct((_G, 1), jnp.float32),
        scratch_shapes=[
            pltpu.VMEM((_G, _F), jnp.float32),
            pltpu.VMEM((_G, _F), jnp.float32),
            pltpu.VMEM((_G, 1), jnp.float32),
            pltpu.VMEM((1, _F), jnp.float32),
            pltpu.VMEM((1, 1), jnp.int32),
        ],
    )(accp, g, dis, b, bcol, brow, bnext, mol, fWa, fWb, fWc, fb, f2W, f2b)


def kernel(x, edge_index, batch, mol_features, W1, b1, W2, b2, W3, b3,
           fc1_W, fc1_b, fc2_W, fc2_b):
    src = edge_index[0].astype(jnp.int32)
    dst = edge_index[1].astype(jnp.int32)
    pad = _E_PAD - _E
    src_p = jnp.concatenate([src, jnp.zeros((pad,), jnp.int32)]).reshape(_TOT_CHUNKS, _CHUNK)
    dst_p = jnp.concatenate([dst, jnp.full((pad,), _N, jnp.int32)]).reshape(_TOT_CHUNKS, _CHUNK)
    bat = batch.astype(jnp.int32)
    bcol = bat.reshape(_N, 1)
    brow = bat.reshape(_NFB, 1, _FBLK)
    bnext = jnp.concatenate([bat[1:], jnp.full((1,), -1, jnp.int32)]).reshape(_NFB, 1, _FBLK)

    degp = _sc_degree(dst_p)[:, :_N, 0:1]
    g1, dis = _tc_prep(x, degp, W1)
    acc1 = _sc_scatter(g1, src_p, dst_p)[:, :_N, :]
    g2 = _tc_mid(acc1, g1, dis, b1.reshape(1, _F), W2)
    acc2 = _sc_scatter(g2, src_p, dst_p)[:, :_N, :]
    g3 = _tc_mid(acc2, g2, dis, b2.reshape(1, _F), W3)
    acc3 = _sc_scatter(g3, src_p, dst_p)[:, :_N, :]
    out = _tc_final(acc3, g3, dis, b3.reshape(1, _F), bcol, brow, bnext,
                    mol_features, fc1_W[:_F], fc1_W[_F:2 * _F], fc1_W[2 * _F:],
                    fc1_b.reshape(1, _F), fc2_W, fc2_b.reshape(1, 1))
    return out[:, 0]


# speedup vs baseline: 2.2983x; 1.0661x over previous
import functools

import jax
import jax.numpy as jnp
from jax import lax
from jax.experimental import pallas as pl
from jax.experimental.pallas import tpu as pltpu
from jax.experimental.pallas import tpu_sc as plsc

_N = 10000
_E = 320000
_G = 200
_F = 64

_NC = 2
_NS = 16
_NW = _NC * _NS
_CHUNK = 128
_A = 80
_B = 80
_TOT_CHUNKS = _NS * (_A + _B)
_E_PAD = _TOT_CHUNKS * _CHUNK
_N_ACC = 10112
_RPT = _N_ACC // _NS

@functools.lru_cache(maxsize=None)
def _mesh():
    return plsc.VectorSubcoreMesh(core_axis_name="c", subcore_axis_name="s",
                                  num_cores=_NC, num_subcores=_NS)


def _sc_degree(dst3):
    k = pl.kernel(
        _sc_degree_body,
        mesh=_mesh(),
        out_type=jax.ShapeDtypeStruct((_NC, _N_ACC, 16), jnp.float32),
        scratch_types=[
            pltpu.VMEM((max(_A, _B), _CHUNK), jnp.int32),
            pltpu.VMEM((_CHUNK, 16), jnp.float32),
            pltpu.VMEM((_RPT, 16), jnp.float32),
            pltpu.VMEM_SHARED((_N_ACC, 16), jnp.float32),
            pltpu.SemaphoreType.DMA,
        ],
        compiler_params=pltpu.CompilerParams(use_tc_tiling_on_sc=False),
    )
    return k(dst3)


def _sc_degree_body(dst_hbm, out_hbm, dsts_v, ones_v, zrows_v, acc_sh, sem):
    cid = lax.axis_index("c")
    sid = lax.axis_index("s")
    one16 = jnp.full((16,), 1.0, jnp.float32)
    zero16 = jnp.zeros((16,), jnp.float32)

    def fill(j, _):
        ones_v[j, :] = one16
        return 0

    lax.fori_loop(0, _CHUNK, fill, 0)

    def zfill(j, _):
        zrows_v[j, :] = zero16
        return 0

    lax.fori_loop(0, _RPT, zfill, 0)

    row_base = sid * _RPT
    pltpu.sync_copy(zrows_v, acc_sh.at[pl.ds(row_base, _RPT)])

    def run(nchunks, start_row):
        if nchunks == 0:
            return
        pltpu.sync_copy(dst_hbm.at[pl.ds(start_row, nchunks)],
                        dsts_v.at[pl.ds(0, nchunks)])
        plsc.subcore_barrier()

        def body(j, _):
            for b in range(8):
                pltpu.async_copy(ones_v, acc_sh.at[dsts_v.at[j * 8 + b]], sem, add=True)
            for b in range(8):
                pltpu.make_async_copy(ones_v, acc_sh.at[dsts_v.at[j * 8 + b]], sem).wait()
            return 0

        lax.fori_loop(0, nchunks // 8, body, 0)

    @pl.when(cid == 0)
    def _():
        run(_A, sid * _A)

    @pl.when(cid == 1)
    def _():
        run(_B, _NS * _A + sid * _B)

    plsc.subcore_barrier()
    pltpu.sync_copy(acc_sh.at[pl.ds(row_base, _RPT)],
                    out_hbm.at[cid, pl.ds(row_base, _RPT)])


_NBUF = 2


def _sc_scatter(g, src3, dst3):
    k = pl.kernel(
        _sc_scatter_body,
        mesh=_mesh(),
        out_type=jax.ShapeDtypeStruct((_NC, _N_ACC, _F), jnp.float32),
        scratch_types=[
            pltpu.VMEM((max(_A, _B), _CHUNK), jnp.int32),
            pltpu.VMEM((max(_A, _B), _CHUNK), jnp.int32),
            pltpu.VMEM((_NBUF, _CHUNK, _F), jnp.float32),
            pltpu.VMEM_SHARED((_N_ACC, _F), jnp.float32),
            pltpu.VMEM_SHARED((_N_ACC, _F), jnp.float32),
            pltpu.SemaphoreType.DMA,
            pltpu.SemaphoreType.DMA,
            pltpu.SemaphoreType.DMA,
            pltpu.SemaphoreType.DMA,
            pltpu.SemaphoreType.DMA,
        ],
        compiler_params=pltpu.CompilerParams(use_tc_tiling_on_sc=False),
    )
    return k(g, src3, dst3)


def _sc_scatter_body(g_hbm, src_hbm, dst_hbm, out_hbm, srcs_v, dsts_v, rows_v,
                     acc_sh, g_sh, gs0, gs1, gs2, gs3, ssem):
    cid = lax.axis_index("c")
    sid = lax.axis_index("s")
    gsems = [gs0, gs1, gs2, gs3]
    zero16 = jnp.zeros((16,), jnp.float32)

    def zfill(j, _):
        for gc in range(_F // 16):
            rows_v[0, j, pl.ds(gc * 16, 16)] = zero16
        return 0

    lax.fori_loop(0, _CHUNK, zfill, 0)

    row_base = sid * _RPT
    for z in range(4):
        pltpu.sync_copy(rows_v.at[0], acc_sh.at[pl.ds(row_base + z * _CHUNK, _CHUNK)])
    pltpu.sync_copy(rows_v.at[0, pl.ds(0, _RPT - 4 * _CHUNK)],
                    acc_sh.at[pl.ds(row_base + 4 * _CHUNK, _RPT - 4 * _CHUNK)])

    @pl.when(sid < _NS - 1)
    def _():
        pltpu.sync_copy(g_hbm.at[pl.ds(row_base, _RPT)],
                        g_sh.at[pl.ds(row_base, _RPT)])

    @pl.when(sid == _NS - 1)
    def _():
        pltpu.sync_copy(g_hbm.at[pl.ds((_NS - 1) * _RPT, _N - (_NS - 1) * _RPT)],
                        g_sh.at[pl.ds((_NS - 1) * _RPT, _N - (_NS - 1) * _RPT)])

    def gstart(c, b):
        pltpu.async_copy(g_sh.at[srcs_v.at[c]], rows_v.at[b], gsems[b])

    def gwait(c, b):
        pltpu.make_async_copy(g_sh.at[srcs_v.at[c]], rows_v.at[b], gsems[b]).wait()

    def sstep(c, b):
        pltpu.async_copy(rows_v.at[b], acc_sh.at[dsts_v.at[c]], ssem, add=True)
        pltpu.make_async_copy(rows_v.at[b], acc_sh.at[dsts_v.at[c]], ssem).wait()

    def run(nchunks, start_row):
        if nchunks == 0:
            return
        pltpu.sync_copy(src_hbm.at[pl.ds(start_row, nchunks)],
                        srcs_v.at[pl.ds(0, nchunks)])
        pltpu.sync_copy(dst_hbm.at[pl.ds(start_row, nchunks)],
                        dsts_v.at[pl.ds(0, nchunks)])
        plsc.subcore_barrier()
        for b in range(_NBUF):
            gstart(b, b)

        def body(j, _):
            for b in range(_NBUF):
                c = j * _NBUF + b
                gwait(c, b)
                sstep(c, b)
                gstart(c + _NBUF, b)
            return 0

        lax.fori_loop(0, nchunks // _NBUF - 1, body, 0)
        for b in range(_NBUF):
            c = nchunks - _NBUF + b
            gwait(c, b)
            sstep(c, b)

    @pl.when(cid == 0)
    def _():
        run(_A, sid * _A)

    @pl.when(cid == 1)
    def _():
        run(_B, _NS * _A + sid * _B)

    plsc.subcore_barrier()
    pltpu.sync_copy(acc_sh.at[pl.ds(row_base, _RPT)],
                    out_hbm.at[cid, pl.ds(row_base, _RPT)])


_BLK = 1000


def _tc_matmul1(x, W1):
    def body(x_ref, W1_ref, u_ref):
        u_ref[...] = jnp.dot(x_ref[...], W1_ref[...],
                             preferred_element_type=jnp.float32)

    return pl.pallas_call(
        body,
        grid=(_N // _BLK,),
        in_specs=[
            pl.BlockSpec((_BLK, 128), lambda i: (i, 0)),
            pl.BlockSpec((128, _F), lambda i: (0, 0)),
        ],
        out_specs=pl.BlockSpec((_BLK, _F), lambda i: (i, 0)),
        out_shape=jax.ShapeDtypeStruct((_N, _F), jnp.float32),
    )(x, W1)


def _tc_scale(degp, u1):
    def body(degp_ref, u_ref, g_ref, dis_ref):
        deg = 1.0 + degp_ref[0, :, 0:1] + degp_ref[1, :, 0:1]
        dis = lax.rsqrt(deg)
        dis_ref[...] = dis
        g_ref[...] = u_ref[...] * dis

    return pl.pallas_call(
        body,
        grid=(_N // _BLK,),
        in_specs=[
            pl.BlockSpec((2, _BLK, 16), lambda i: (0, i, 0)),
            pl.BlockSpec((_BLK, _F), lambda i: (i, 0)),
        ],
        out_specs=[
            pl.BlockSpec((_BLK, _F), lambda i: (i, 0)),
            pl.BlockSpec((_BLK, 1), lambda i: (i, 0)),
        ],
        out_shape=[
            jax.ShapeDtypeStruct((_N, _F), jnp.float32),
            jax.ShapeDtypeStruct((_N, 1), jnp.float32),
        ],
    )(degp, u1)


def _tc_mid(accp, g, dis, b, W):
    def body(accp_ref, g_ref, dis_ref, b_ref, W_ref, gn_ref):
        dis = dis_ref[...]
        h = jnp.maximum(dis * (accp_ref[0] + accp_ref[1] + g_ref[...]) + b_ref[...], 0.0)
        gn_ref[...] = jnp.dot(h, W_ref[...], preferred_element_type=jnp.float32) * dis

    return pl.pallas_call(
        body,
        grid=(_N // _BLK,),
        in_specs=[
            pl.BlockSpec((2, _BLK, _F), lambda i: (0, i, 0)),
            pl.BlockSpec((_BLK, _F), lambda i: (i, 0)),
            pl.BlockSpec((_BLK, 1), lambda i: (i, 0)),
            pl.BlockSpec((1, _F), lambda i: (0, 0)),
            pl.BlockSpec((_F, _F), lambda i: (0, 0)),
        ],
        out_specs=pl.BlockSpec((_BLK, _F), lambda i: (i, 0)),
        out_shape=jax.ShapeDtypeStruct((_N, _F), jnp.float32),
    )(accp, g, dis, b, W)


_FBLK = 400
_NFB = _N // _FBLK


def _tc_final(accp, g, dis, b, bcol, brow, bnext, mol, fWa, fWb, fWc, fb, f2W, f2b):
    def body(accp_ref, g_ref, dis_ref, b_ref, bcol_ref, brow_ref, bnext_ref,
             mol_ref, fWa_ref, fWb_ref, fWc_ref, fb_ref, f2W_ref, f2b_ref,
             out_ref, gmax_s, gsum_s, cnt_s, cm_s, cb_s):
        i = pl.program_id(0)

        @pl.when(i == 0)
        def _():
            gmax_s[...] = jnp.zeros((_G, _F), jnp.float32)
            gsum_s[...] = jnp.zeros((_G, _F), jnp.float32)
            cnt_s[...] = jnp.zeros((_G, 1), jnp.float32)
            cm_s[...] = jnp.zeros((1, _F), jnp.float32)
            cb_s[...] = jnp.full((1, 1), -1, jnp.int32)

        h = jnp.maximum(
            dis_ref[...] * (accp_ref[0] + accp_ref[1] + g_ref[...]) + b_ref[...], 0.0)
        bcol = bcol_ref[...]
        m = h
        for k in range(9):
            sh = 1 << k
            m_sh = jnp.concatenate(
                [jnp.zeros((sh, _F), jnp.float32), m[:_FBLK - sh]], axis=0)
            b_sh = jnp.concatenate(
                [jnp.full((sh, 1), -1, jnp.int32), bcol[:_FBLK - sh]], axis=0)
            m = jnp.where(bcol == b_sh, jnp.maximum(m, m_sh), m)
        m = jnp.where(bcol == cb_s[...], jnp.maximum(m, cm_s[...]), m)
        cm_s[...] = m[_FBLK - 1:_FBLK]
        cb_s[...] = bcol[_FBLK - 1:_FBLK]

        brow = brow_ref[0]
        gids = lax.broadcasted_iota(jnp.int32, (_G, _FBLK), 0)
        member = (gids == brow)
        soh = member.astype(jnp.float32)
        eoh = (member & (brow != bnext_ref[0])).astype(jnp.float32)
        gmax_s[...] += jnp.dot(eoh, m, preferred_element_type=jnp.float32)
        gsum_s[...] += jnp.dot(soh, h, preferred_element_type=jnp.float32)
        cnt_s[...] += jnp.sum(soh, axis=1, keepdims=True)

        @pl.when(i == _NFB - 1)
        def _():
            gmean = gsum_s[...] / jnp.maximum(cnt_s[...], 1.0)
            cc = (jnp.dot(gmax_s[...], fWa_ref[...], preferred_element_type=jnp.float32)
                  + jnp.dot(gmean, fWb_ref[...], preferred_element_type=jnp.float32)
                  + jnp.dot(mol_ref[...], fWc_ref[...], preferred_element_type=jnp.float32)
                  + fb_ref[...])
            cc = jnp.maximum(cc, 0.0)
            out_ref[...] = (jnp.dot(cc, f2W_ref[...], preferred_element_type=jnp.float32)
                            + f2b_ref[...])

    return pl.pallas_call(
        body,
        grid=(_NFB,),
        in_specs=[
            pl.BlockSpec((2, _FBLK, _F), lambda i: (0, i, 0)),
            pl.BlockSpec((_FBLK, _F), lambda i: (i, 0)),
            pl.BlockSpec((_FBLK, 1), lambda i: (i, 0)),
            pl.BlockSpec((1, _F), lambda i: (0, 0)),
            pl.BlockSpec((_FBLK, 1), lambda i: (i, 0)),
            pl.BlockSpec((1, 1, _FBLK), lambda i: (i, 0, 0)),
            pl.BlockSpec((1, 1, _FBLK), lambda i: (i, 0, 0)),
            pl.BlockSpec((_G, 16), lambda i: (0, 0)),
            pl.BlockSpec((_F, _F), lambda i: (0, 0)),
            pl.BlockSpec((_F, _F), lambda i: (0, 0)),
            pl.BlockSpec((16, _F), lambda i: (0, 0)),
            pl.BlockSpec((1, _F), lambda i: (0, 0)),
            pl.BlockSpec((_F, 1), lambda i: (0, 0)),
            pl.BlockSpec((1, 1), lambda i: (0, 0)),
        ],
        out_specs=pl.BlockSpec((_G, 1), lambda i: (0, 0)),
        out_shape=jax.ShapeDtypeStruct((_G, 1), jnp.float32),
        scratch_shapes=[
            pltpu.VMEM((_G, _F), jnp.float32),
            pltpu.VMEM((_G, _F), jnp.float32),
            pltpu.VMEM((_G, 1), jnp.float32),
            pltpu.VMEM((1, _F), jnp.float32),
            pltpu.VMEM((1, 1), jnp.int32),
        ],
    )(accp, g, dis, b, bcol, brow, bnext, mol, fWa, fWb, fWc, fb, f2W, f2b)


def kernel(x, edge_index, batch, mol_features, W1, b1, W2, b2, W3, b3,
           fc1_W, fc1_b, fc2_W, fc2_b):
    src = edge_index[0].astype(jnp.int32)
    dst = edge_index[1].astype(jnp.int32)
    pad = _E_PAD - _E
    src_p = jnp.concatenate([src, jnp.zeros((pad,), jnp.int32)]).reshape(_TOT_CHUNKS, _CHUNK)
    dst_p = jnp.concatenate([dst, jnp.full((pad,), _N, jnp.int32)]).reshape(_TOT_CHUNKS, _CHUNK)
    bat = batch.astype(jnp.int32)
    bcol = bat.reshape(_N, 1)
    brow = bat.reshape(_NFB, 1, _FBLK)
    bnext = jnp.concatenate([bat[1:], jnp.full((1,), -1, jnp.int32)]).reshape(_NFB, 1, _FBLK)

    u1 = _tc_matmul1(x, W1)
    degp = _sc_degree(dst_p)
    g1, dis = _tc_scale(degp, u1)
    acc1 = _sc_scatter(g1, src_p, dst_p)
    g2 = _tc_mid(acc1, g1, dis, b1.reshape(1, _F), W2)
    acc2 = _sc_scatter(g2, src_p, dst_p)
    g3 = _tc_mid(acc2, g2, dis, b2.reshape(1, _F), W3)
    acc3 = _sc_scatter(g3, src_p, dst_p)
    out = _tc_final(acc3, g3, dis, b3.reshape(1, _F), bcol, brow, bnext,
                    mol_features, fc1_W[:_F], fc1_W[_F:2 * _F], fc1_W[2 * _F:],
                    fc1_b.reshape(1, _F), fc2_W, fc2_b.reshape(1, 1))
    return out[:, 0]


# speedup vs baseline: 2.3262x; 1.0121x over previous
import functools

import jax
import jax.numpy as jnp
from jax import lax
from jax.experimental import pallas as pl
from jax.experimental.pallas import tpu as pltpu
from jax.experimental.pallas import tpu_sc as plsc

_N = 10000
_E = 320000
_G = 200
_F = 64

_NC = 2
_NS = 16
_NW = _NC * _NS
_CHUNK = 128
_A = 80
_B = 80
_TOT_CHUNKS = _NS * (_A + _B)
_E_PAD = _TOT_CHUNKS * _CHUNK
_N_ACC = 10112
_RPT = _N_ACC // _NS

@functools.lru_cache(maxsize=None)
def _mesh():
    return plsc.VectorSubcoreMesh(core_axis_name="c", subcore_axis_name="s",
                                  num_cores=_NC, num_subcores=_NS)


def _sc_degree(dst3):
    k = pl.kernel(
        _sc_degree_body,
        mesh=_mesh(),
        out_type=jax.ShapeDtypeStruct((_NC, _N_ACC, 16), jnp.float32),
        scratch_types=[
            pltpu.VMEM((max(_A, _B), _CHUNK), jnp.int32),
            pltpu.VMEM((_CHUNK, 16), jnp.float32),
            pltpu.VMEM((_RPT, 16), jnp.float32),
            pltpu.VMEM_SHARED((_N_ACC, 16), jnp.float32),
            pltpu.SemaphoreType.DMA,
        ],
        compiler_params=pltpu.CompilerParams(use_tc_tiling_on_sc=False),
    )
    return k(dst3)


def _sc_degree_body(dst_hbm, out_hbm, dsts_v, ones_v, zrows_v, acc_sh, sem):
    cid = lax.axis_index("c")
    sid = lax.axis_index("s")
    one16 = jnp.full((16,), 1.0, jnp.float32)
    zero16 = jnp.zeros((16,), jnp.float32)

    def fill(j, _):
        ones_v[j, :] = one16
        return 0

    lax.fori_loop(0, _CHUNK, fill, 0)

    def zfill(j, _):
        zrows_v[j, :] = zero16
        return 0

    lax.fori_loop(0, _RPT, zfill, 0)

    row_base = sid * _RPT
    pltpu.sync_copy(zrows_v, acc_sh.at[pl.ds(row_base, _RPT)])

    def run(nchunks, start_row):
        if nchunks == 0:
            return
        pltpu.sync_copy(dst_hbm.at[pl.ds(start_row, nchunks)],
                        dsts_v.at[pl.ds(0, nchunks)])
        plsc.subcore_barrier()

        def body(j, _):
            for b in range(8):
                pltpu.async_copy(ones_v, acc_sh.at[dsts_v.at[j * 8 + b]], sem, add=True)
            for b in range(8):
                pltpu.make_async_copy(ones_v, acc_sh.at[dsts_v.at[j * 8 + b]], sem).wait()
            return 0

        lax.fori_loop(0, nchunks // 8, body, 0)

    @pl.when(cid == 0)
    def _():
        run(_A, sid * _A)

    @pl.when(cid == 1)
    def _():
        run(_B, _NS * _A + sid * _B)

    plsc.subcore_barrier()
    pltpu.sync_copy(acc_sh.at[pl.ds(row_base, _RPT)],
                    out_hbm.at[cid, pl.ds(row_base, _RPT)])


_NBUF = 3


def _sc_scatter(g, src3, dst3):
    k = pl.kernel(
        _sc_scatter_body,
        mesh=_mesh(),
        out_type=jax.ShapeDtypeStruct((_NC, _N_ACC, _F), jnp.float32),
        scratch_types=[
            pltpu.VMEM((max(_A, _B), _CHUNK), jnp.int32),
            pltpu.VMEM((max(_A, _B), _CHUNK), jnp.int32),
            pltpu.VMEM((_NBUF, _CHUNK, _F), jnp.float32),
            pltpu.VMEM_SHARED((_N_ACC, _F), jnp.float32),
            pltpu.VMEM_SHARED((_N_ACC, _F), jnp.float32),
            pltpu.SemaphoreType.DMA,
            pltpu.SemaphoreType.DMA,
            pltpu.SemaphoreType.DMA,
            pltpu.SemaphoreType.DMA,
            pltpu.SemaphoreType.DMA,
        ],
        compiler_params=pltpu.CompilerParams(use_tc_tiling_on_sc=False),
    )
    return k(g, src3, dst3)


def _sc_scatter_body(g_hbm, src_hbm, dst_hbm, out_hbm, srcs_v, dsts_v, rows_v,
                     acc_sh, g_sh, gs0, gs1, gs2, gs3, ssem):
    cid = lax.axis_index("c")
    sid = lax.axis_index("s")
    gsems = [gs0, gs1, gs2, gs3]
    zero16 = jnp.zeros((16,), jnp.float32)

    def zfill(j, _):
        for gc in range(_F // 16):
            rows_v[0, j, pl.ds(gc * 16, 16)] = zero16
        return 0

    lax.fori_loop(0, _CHUNK, zfill, 0)

    row_base = sid * _RPT
    for z in range(4):
        pltpu.sync_copy(rows_v.at[0], acc_sh.at[pl.ds(row_base + z * _CHUNK, _CHUNK)])
    pltpu.sync_copy(rows_v.at[0, pl.ds(0, _RPT - 4 * _CHUNK)],
                    acc_sh.at[pl.ds(row_base + 4 * _CHUNK, _RPT - 4 * _CHUNK)])

    @pl.when(sid < _NS - 1)
    def _():
        pltpu.sync_copy(g_hbm.at[pl.ds(row_base, _RPT)],
                        g_sh.at[pl.ds(row_base, _RPT)])

    @pl.when(sid == _NS - 1)
    def _():
        pltpu.sync_copy(g_hbm.at[pl.ds((_NS - 1) * _RPT, _N - (_NS - 1) * _RPT)],
                        g_sh.at[pl.ds((_NS - 1) * _RPT, _N - (_NS - 1) * _RPT)])

    def gstart(c, b):
        pltpu.async_copy(g_sh.at[srcs_v.at[c]], rows_v.at[b], gsems[b])

    def gwait(c, b):
        pltpu.make_async_copy(g_sh.at[srcs_v.at[c]], rows_v.at[b], gsems[b]).wait()

    def sstep(c, b):
        pltpu.async_copy(rows_v.at[b], acc_sh.at[dsts_v.at[c]], ssem, add=True)
        pltpu.make_async_copy(rows_v.at[b], acc_sh.at[dsts_v.at[c]], ssem).wait()

    def run(nchunks, start_row):
        if nchunks == 0:
            return
        pltpu.sync_copy(src_hbm.at[pl.ds(start_row, nchunks)],
                        srcs_v.at[pl.ds(0, nchunks)])
        pltpu.sync_copy(dst_hbm.at[pl.ds(start_row, nchunks)],
                        dsts_v.at[pl.ds(0, nchunks)])
        plsc.subcore_barrier()
        for b in range(_NBUF):
            gstart(b, b)

        def body(j, _):
            for b in range(_NBUF):
                c = j * _NBUF + b
                gwait(c, b)
                sstep(c, b)
                gstart(c + _NBUF, b)
            return 0

        nfull = (nchunks - _NBUF) // _NBUF
        lax.fori_loop(0, nfull, body, 0)
        for c in range(nfull * _NBUF, nchunks):
            b = c % _NBUF
            gwait(c, b)
            sstep(c, b)
            if c + _NBUF < nchunks:
                gstart(c + _NBUF, b)

    @pl.when(cid == 0)
    def _():
        run(_A, sid * _A)

    @pl.when(cid == 1)
    def _():
        run(_B, _NS * _A + sid * _B)

    plsc.subcore_barrier()
    pltpu.sync_copy(acc_sh.at[pl.ds(row_base, _RPT)],
                    out_hbm.at[cid, pl.ds(row_base, _RPT)])


_BLK = 1000


def _tc_matmul1(x, W1):
    def body(x_ref, W1_ref, u_ref):
        u_ref[...] = jnp.dot(x_ref[...], W1_ref[...],
                             preferred_element_type=jnp.float32)

    return pl.pallas_call(
        body,
        grid=(_N // _BLK,),
        in_specs=[
            pl.BlockSpec((_BLK, 128), lambda i: (i, 0)),
            pl.BlockSpec((128, _F), lambda i: (0, 0)),
        ],
        out_specs=pl.BlockSpec((_BLK, _F), lambda i: (i, 0)),
        out_shape=jax.ShapeDtypeStruct((_N, _F), jnp.float32),
    )(x, W1)


def _tc_scale(degp, u1):
    def body(degp_ref, u_ref, g_ref, dis_ref):
        deg = 1.0 + degp_ref[0, :, 0:1] + degp_ref[1, :, 0:1]
        dis = lax.rsqrt(deg)
        dis_ref[...] = dis
        g_ref[...] = u_ref[...] * dis

    return pl.pallas_call(
        body,
        grid=(_N // _BLK,),
        in_specs=[
            pl.BlockSpec((2, _BLK, 16), lambda i: (0, i, 0)),
            pl.BlockSpec((_BLK, _F), lambda i: (i, 0)),
        ],
        out_specs=[
            pl.BlockSpec((_BLK, _F), lambda i: (i, 0)),
            pl.BlockSpec((_BLK, 1), lambda i: (i, 0)),
        ],
        out_shape=[
            jax.ShapeDtypeStruct((_N, _F), jnp.float32),
            jax.ShapeDtypeStruct((_N, 1), jnp.float32),
        ],
    )(degp, u1)


def _tc_mid(accp, g, dis, b, W):
    def body(accp_ref, g_ref, dis_ref, b_ref, W_ref, gn_ref):
        dis = dis_ref[...]
        h = jnp.maximum(dis * (accp_ref[0] + accp_ref[1] + g_ref[...]) + b_ref[...], 0.0)
        gn_ref[...] = jnp.dot(h, W_ref[...], preferred_element_type=jnp.float32) * dis

    return pl.pallas_call(
        body,
        grid=(_N // _BLK,),
        in_specs=[
            pl.BlockSpec((2, _BLK, _F), lambda i: (0, i, 0)),
            pl.BlockSpec((_BLK, _F), lambda i: (i, 0)),
            pl.BlockSpec((_BLK, 1), lambda i: (i, 0)),
            pl.BlockSpec((1, _F), lambda i: (0, 0)),
            pl.BlockSpec((_F, _F), lambda i: (0, 0)),
        ],
        out_specs=pl.BlockSpec((_BLK, _F), lambda i: (i, 0)),
        out_shape=jax.ShapeDtypeStruct((_N, _F), jnp.float32),
    )(accp, g, dis, b, W)


_FBLK = 400
_NFB = _N // _FBLK


def _tc_final(accp, g, dis, b, bcol, brow, bnext, mol, fWa, fWb, fWc, fb, f2W, f2b):
    def body(accp_ref, g_ref, dis_ref, b_ref, bcol_ref, brow_ref, bnext_ref,
             mol_ref, fWa_ref, fWb_ref, fWc_ref, fb_ref, f2W_ref, f2b_ref,
             out_ref, gmax_s, gsum_s, cnt_s, cm_s, cb_s):
        i = pl.program_id(0)

        @pl.when(i == 0)
        def _():
            gmax_s[...] = jnp.zeros((_G, _F), jnp.float32)
            gsum_s[...] = jnp.zeros((_G, _F), jnp.float32)
            cnt_s[...] = jnp.zeros((_G, 1), jnp.float32)
            cm_s[...] = jnp.zeros((1, _F), jnp.float32)
            cb_s[...] = jnp.full((1, 1), -1, jnp.int32)

        h = jnp.maximum(
            dis_ref[...] * (accp_ref[0] + accp_ref[1] + g_ref[...]) + b_ref[...], 0.0)
        bcol = bcol_ref[...]
        m = h
        for k in range(9):
            sh = 1 << k
            m_sh = jnp.concatenate(
                [jnp.zeros((sh, _F), jnp.float32), m[:_FBLK - sh]], axis=0)
            b_sh = jnp.concatenate(
                [jnp.full((sh, 1), -1, jnp.int32), bcol[:_FBLK - sh]], axis=0)
            m = jnp.where(bcol == b_sh, jnp.maximum(m, m_sh), m)
        m = jnp.where(bcol == cb_s[...], jnp.maximum(m, cm_s[...]), m)
        cm_s[...] = m[_FBLK - 1:_FBLK]
        cb_s[...] = bcol[_FBLK - 1:_FBLK]

        brow = brow_ref[0]
        gids = lax.broadcasted_iota(jnp.int32, (_G, _FBLK), 0)
        member = (gids == brow)
        soh = member.astype(jnp.float32)
        eoh = (member & (brow != bnext_ref[0])).astype(jnp.float32)
        gmax_s[...] += jnp.dot(eoh, m, preferred_element_type=jnp.float32)
        gsum_s[...] += jnp.dot(soh, h, preferred_element_type=jnp.float32)
        cnt_s[...] += jnp.sum(soh, axis=1, keepdims=True)

        @pl.when(i == _NFB - 1)
        def _():
            gmean = gsum_s[...] / jnp.maximum(cnt_s[...], 1.0)
            cc = (jnp.dot(gmax_s[...], fWa_ref[...], preferred_element_type=jnp.float32)
                  + jnp.dot(gmean, fWb_ref[...], preferred_element_type=jnp.float32)
                  + jnp.dot(mol_ref[...], fWc_ref[...], preferred_element_type=jnp.float32)
                  + fb_ref[...])
            cc = jnp.maximum(cc, 0.0)
            out_ref[...] = (jnp.dot(cc, f2W_ref[...], preferred_element_type=jnp.float32)
                            + f2b_ref[...])

    return pl.pallas_call(
        body,
        grid=(_NFB,),
        in_specs=[
            pl.BlockSpec((2, _FBLK, _F), lambda i: (0, i, 0)),
            pl.BlockSpec((_FBLK, _F), lambda i: (i, 0)),
            pl.BlockSpec((_FBLK, 1), lambda i: (i, 0)),
            pl.BlockSpec((1, _F), lambda i: (0, 0)),
            pl.BlockSpec((_FBLK, 1), lambda i: (i, 0)),
            pl.BlockSpec((1, 1, _FBLK), lambda i: (i, 0, 0)),
            pl.BlockSpec((1, 1, _FBLK), lambda i: (i, 0, 0)),
            pl.BlockSpec((_G, 16), lambda i: (0, 0)),
            pl.BlockSpec((_F, _F), lambda i: (0, 0)),
            pl.BlockSpec((_F, _F), lambda i: (0, 0)),
            pl.BlockSpec((16, _F), lambda i: (0, 0)),
            pl.BlockSpec((1, _F), lambda i: (0, 0)),
            pl.BlockSpec((_F, 1), lambda i: (0, 0)),
            pl.BlockSpec((1, 1), lambda i: (0, 0)),
        ],
        out_specs=pl.BlockSpec((_G, 1), lambda i: (0, 0)),
        out_shape=jax.ShapeDtypeStruct((_G, 1), jnp.float32),
        scratch_shapes=[
            pltpu.VMEM((_G, _F), jnp.float32),
            pltpu.VMEM((_G, _F), jnp.float32),
            pltpu.VMEM((_G, 1), jnp.float32),
            pltpu.VMEM((1, _F), jnp.float32),
            pltpu.VMEM((1, 1), jnp.int32),
        ],
    )(accp, g, dis, b, bcol, brow, bnext, mol, fWa, fWb, fWc, fb, f2W, f2b)


def kernel(x, edge_index, batch, mol_features, W1, b1, W2, b2, W3, b3,
           fc1_W, fc1_b, fc2_W, fc2_b):
    src = edge_index[0].astype(jnp.int32)
    dst = edge_index[1].astype(jnp.int32)
    pad = _E_PAD - _E
    src_p = jnp.concatenate([src, jnp.zeros((pad,), jnp.int32)]).reshape(_TOT_CHUNKS, _CHUNK)
    dst_p = jnp.concatenate([dst, jnp.full((pad,), _N, jnp.int32)]).reshape(_TOT_CHUNKS, _CHUNK)
    bat = batch.astype(jnp.int32)
    bcol = bat.reshape(_N, 1)
    brow = bat.reshape(_NFB, 1, _FBLK)
    bnext = jnp.concatenate([bat[1:], jnp.full((1,), -1, jnp.int32)]).reshape(_NFB, 1, _FBLK)

    u1 = _tc_matmul1(x, W1)
    degp = _sc_degree(dst_p)
    g1, dis = _tc_scale(degp, u1)
    acc1 = _sc_scatter(g1, src_p, dst_p)
    g2 = _tc_mid(acc1, g1, dis, b1.reshape(1, _F), W2)
    acc2 = _sc_scatter(g2, src_p, dst_p)
    g3 = _tc_mid(acc2, g2, dis, b2.reshape(1, _F), W3)
    acc3 = _sc_scatter(g3, src_p, dst_p)
    out = _tc_final(acc3, g3, dis, b3.reshape(1, _F), bcol, brow, bnext,
                    mol_features, fc1_W[:_F], fc1_W[_F:2 * _F], fc1_W[2 * _F:],
                    fc1_b.reshape(1, _F), fc2_W, fc2_b.reshape(1, 1))
    return out[:, 0]


# speedup vs baseline: 2.3545x; 1.0122x over previous
import functools

import jax
import jax.numpy as jnp
from jax import lax
from jax.experimental import pallas as pl
from jax.experimental.pallas import tpu as pltpu
from jax.experimental.pallas import tpu_sc as plsc

_N = 10000
_E = 320000
_G = 200
_F = 64

_NC = 2
_NS = 16
_NW = _NC * _NS
_CHUNK = 128
_A = 80
_B = 80
_TOT_CHUNKS = _NS * (_A + _B)
_E_PAD = _TOT_CHUNKS * _CHUNK
_N_ACC = 10112
_RPT = _N_ACC // _NS

@functools.lru_cache(maxsize=None)
def _mesh():
    return plsc.VectorSubcoreMesh(core_axis_name="c", subcore_axis_name="s",
                                  num_cores=_NC, num_subcores=_NS)


def _sc_degree(dst3):
    k = pl.kernel(
        _sc_degree_body,
        mesh=_mesh(),
        out_type=jax.ShapeDtypeStruct((_NC, _N_ACC, 16), jnp.float32),
        scratch_types=[
            pltpu.VMEM((max(_A, _B), _CHUNK), jnp.int32),
            pltpu.VMEM((_CHUNK, 16), jnp.float32),
            pltpu.VMEM((_RPT, 16), jnp.float32),
            pltpu.VMEM_SHARED((_N_ACC, 16), jnp.float32),
            pltpu.SemaphoreType.DMA,
        ],
        compiler_params=pltpu.CompilerParams(use_tc_tiling_on_sc=False),
    )
    return k(dst3)


def _sc_degree_body(dst_hbm, out_hbm, dsts_v, ones_v, zrows_v, acc_sh, sem):
    cid = lax.axis_index("c")
    sid = lax.axis_index("s")
    one16 = jnp.full((16,), 1.0, jnp.float32)
    zero16 = jnp.zeros((16,), jnp.float32)

    def fill(j, _):
        ones_v[j, :] = one16
        return 0

    lax.fori_loop(0, _CHUNK, fill, 0)

    def zfill(j, _):
        zrows_v[j, :] = zero16
        return 0

    lax.fori_loop(0, _RPT, zfill, 0)

    row_base = sid * _RPT
    pltpu.sync_copy(zrows_v, acc_sh.at[pl.ds(row_base, _RPT)])

    def run(nchunks, start_row):
        if nchunks == 0:
            return
        pltpu.sync_copy(dst_hbm.at[pl.ds(start_row, nchunks)],
                        dsts_v.at[pl.ds(0, nchunks)])
        plsc.subcore_barrier()

        def body(j, _):
            for b in range(8):
                pltpu.async_copy(ones_v, acc_sh.at[dsts_v.at[j * 8 + b]], sem, add=True)
            for b in range(8):
                pltpu.make_async_copy(ones_v, acc_sh.at[dsts_v.at[j * 8 + b]], sem).wait()
            return 0

        lax.fori_loop(0, nchunks // 8, body, 0)

    @pl.when(cid == 0)
    def _():
        run(_A, sid * _A)

    @pl.when(cid == 1)
    def _():
        run(_B, _NS * _A + sid * _B)

    plsc.subcore_barrier()
    pltpu.sync_copy(acc_sh.at[pl.ds(row_base, _RPT)],
                    out_hbm.at[cid, pl.ds(row_base, _RPT)])


_NBUF = 3


def _sc_scatter(g, src3, dst3):
    k = pl.kernel(
        _sc_scatter_body,
        mesh=_mesh(),
        out_type=jax.ShapeDtypeStruct((_NC, _N_ACC, _F), jnp.float32),
        scratch_types=[
            pltpu.VMEM((max(_A, _B), _CHUNK), jnp.int32),
            pltpu.VMEM((max(_A, _B), _CHUNK), jnp.int32),
            pltpu.VMEM((_NBUF, _CHUNK, _F), jnp.float32),
            pltpu.VMEM_SHARED((_N_ACC, _F), jnp.float32),
            pltpu.VMEM_SHARED((_N_ACC, _F), jnp.float32),
            pltpu.SemaphoreType.DMA,
            pltpu.SemaphoreType.DMA,
            pltpu.SemaphoreType.DMA,
            pltpu.SemaphoreType.DMA,
            pltpu.SemaphoreType.DMA,
        ],
        compiler_params=pltpu.CompilerParams(use_tc_tiling_on_sc=False),
    )
    return k(g, src3, dst3)


def _sc_scatter_body(g_hbm, src_hbm, dst_hbm, out_hbm, srcs_v, dsts_v, rows_v,
                     acc_sh, g_sh, gs0, gs1, gs2, gs3, ssem):
    cid = lax.axis_index("c")
    sid = lax.axis_index("s")
    gsems = [gs0, gs1, gs2, gs3]
    zero16 = jnp.zeros((16,), jnp.float32)

    def zfill(j, _):
        for gc in range(_F // 16):
            rows_v[0, j, pl.ds(gc * 16, 16)] = zero16
        return 0

    lax.fori_loop(0, _CHUNK, zfill, 0)

    row_base = sid * _RPT
    for z in range(4):
        pltpu.sync_copy(rows_v.at[0], acc_sh.at[pl.ds(row_base + z * _CHUNK, _CHUNK)])
    pltpu.sync_copy(rows_v.at[0, pl.ds(0, _RPT - 4 * _CHUNK)],
                    acc_sh.at[pl.ds(row_base + 4 * _CHUNK, _RPT - 4 * _CHUNK)])

    @pl.when(sid < _NS - 1)
    def _():
        pltpu.sync_copy(g_hbm.at[pl.ds(row_base, _RPT)],
                        g_sh.at[pl.ds(row_base, _RPT)])

    @pl.when(sid == _NS - 1)
    def _():
        pltpu.sync_copy(g_hbm.at[pl.ds((_NS - 1) * _RPT, _N - (_NS - 1) * _RPT)],
                        g_sh.at[pl.ds((_NS - 1) * _RPT, _N - (_NS - 1) * _RPT)])

    def gstart(c, b):
        pltpu.async_copy(g_sh.at[srcs_v.at[c]], rows_v.at[b], gsems[b])

    def gwait(c, b):
        pltpu.make_async_copy(g_sh.at[srcs_v.at[c]], rows_v.at[b], gsems[b]).wait()

    def sstep(c, b):
        pltpu.async_copy(rows_v.at[b], acc_sh.at[dsts_v.at[c]], ssem, add=True)
        pltpu.make_async_copy(rows_v.at[b], acc_sh.at[dsts_v.at[c]], ssem).wait()

    def run(nchunks, start_row):
        if nchunks == 0:
            return
        pltpu.sync_copy(src_hbm.at[pl.ds(start_row, nchunks)],
                        srcs_v.at[pl.ds(0, nchunks)])
        pltpu.sync_copy(dst_hbm.at[pl.ds(start_row, nchunks)],
                        dsts_v.at[pl.ds(0, nchunks)])
        plsc.subcore_barrier()
        for b in range(_NBUF):
            gstart(b, b)

        def body(j, _):
            for b in range(_NBUF):
                c = j * _NBUF + b
                gwait(c, b)
                sstep(c, b)
                gstart(c + _NBUF, b)
            return 0

        nfull = (nchunks - _NBUF) // _NBUF
        lax.fori_loop(0, nfull, body, 0)
        for c in range(nfull * _NBUF, nchunks):
            b = c % _NBUF
            gwait(c, b)
            sstep(c, b)
            if c + _NBUF < nchunks:
                gstart(c + _NBUF, b)

    @pl.when(cid == 0)
    def _():
        run(_A, sid * _A)

    @pl.when(cid == 1)
    def _():
        run(_B, _NS * _A + sid * _B)

    plsc.subcore_barrier()
    pltpu.sync_copy(acc_sh.at[pl.ds(row_base, _RPT)],
                    out_hbm.at[cid, pl.ds(row_base, _RPT)])


_BLK = 1000


def _tc_matmul1(x, W1):
    def body(x_ref, W1_ref, u_ref):
        u_ref[...] = jnp.dot(x_ref[...], W1_ref[...],
                             preferred_element_type=jnp.float32)

    return pl.pallas_call(
        body,
        grid=(_N // _BLK,),
        in_specs=[
            pl.BlockSpec((_BLK, 128), lambda i: (i, 0)),
            pl.BlockSpec((128, _F), lambda i: (0, 0)),
        ],
        out_specs=pl.BlockSpec((_BLK, _F), lambda i: (i, 0)),
        out_shape=jax.ShapeDtypeStruct((_N, _F), jnp.float32),
    )(x, W1)


def _tc_scale(degp, u1):
    def body(degp_ref, u_ref, g_ref, dis_ref):
        deg = 1.0 + degp_ref[0, :, 0:1] + degp_ref[1, :, 0:1]
        dis = lax.rsqrt(deg)
        dis_ref[...] = dis
        g_ref[...] = u_ref[...] * dis

    return pl.pallas_call(
        body,
        grid=(_N // _BLK,),
        in_specs=[
            pl.BlockSpec((2, _BLK, 16), lambda i: (0, i, 0)),
            pl.BlockSpec((_BLK, _F), lambda i: (i, 0)),
        ],
        out_specs=[
            pl.BlockSpec((_BLK, _F), lambda i: (i, 0)),
            pl.BlockSpec((_BLK, 1), lambda i: (i, 0)),
        ],
        out_shape=[
            jax.ShapeDtypeStruct((_N, _F), jnp.float32),
            jax.ShapeDtypeStruct((_N, 1), jnp.float32),
        ],
    )(degp, u1)


def _tc_mid(accp, g, dis, b, W):
    def body(accp_ref, g_ref, dis_ref, b_ref, W_ref, gn_ref):
        dis = dis_ref[...]
        h = jnp.maximum(dis * (accp_ref[0] + accp_ref[1] + g_ref[...]) + b_ref[...], 0.0)
        gn_ref[...] = jnp.dot(h, W_ref[...], preferred_element_type=jnp.float32) * dis

    return pl.pallas_call(
        body,
        grid=(_N // _BLK,),
        in_specs=[
            pl.BlockSpec((2, _BLK, _F), lambda i: (0, i, 0)),
            pl.BlockSpec((_BLK, _F), lambda i: (i, 0)),
            pl.BlockSpec((_BLK, 1), lambda i: (i, 0)),
            pl.BlockSpec((1, _F), lambda i: (0, 0)),
            pl.BlockSpec((_F, _F), lambda i: (0, 0)),
        ],
        out_specs=pl.BlockSpec((_BLK, _F), lambda i: (i, 0)),
        out_shape=jax.ShapeDtypeStruct((_N, _F), jnp.float32),
    )(accp, g, dis, b, W)


_FBLK = 1000
_NFB = _N // _FBLK


def _tc_final(accp, g, dis, b, bcol, brow, bnext, mol, fWa, fWb, fWc, fb, f2W, f2b):
    def body(accp_ref, g_ref, dis_ref, b_ref, bcol_ref, brow_ref, bnext_ref,
             mol_ref, fWa_ref, fWb_ref, fWc_ref, fb_ref, f2W_ref, f2b_ref,
             out_ref, gmax_s, gsum_s, cnt_s, cm_s, cb_s):
        i = pl.program_id(0)

        @pl.when(i == 0)
        def _():
            gmax_s[...] = jnp.zeros((_G, _F), jnp.float32)
            gsum_s[...] = jnp.zeros((_G, _F), jnp.float32)
            cnt_s[...] = jnp.zeros((_G, 1), jnp.float32)
            cm_s[...] = jnp.zeros((1, _F), jnp.float32)
            cb_s[...] = jnp.full((1, 1), -1, jnp.int32)

        h = jnp.maximum(
            dis_ref[...] * (accp_ref[0] + accp_ref[1] + g_ref[...]) + b_ref[...], 0.0)
        bcol = bcol_ref[...]
        m = h
        for k in range(10):
            sh = 1 << k
            m_sh = jnp.concatenate(
                [jnp.zeros((sh, _F), jnp.float32), m[:_FBLK - sh]], axis=0)
            b_sh = jnp.concatenate(
                [jnp.full((sh, 1), -1, jnp.int32), bcol[:_FBLK - sh]], axis=0)
            m = jnp.where(bcol == b_sh, jnp.maximum(m, m_sh), m)
        m = jnp.where(bcol == cb_s[...], jnp.maximum(m, cm_s[...]), m)
        cm_s[...] = m[_FBLK - 1:_FBLK]
        cb_s[...] = bcol[_FBLK - 1:_FBLK]

        brow = brow_ref[0]
        gids = lax.broadcasted_iota(jnp.int32, (_G, _FBLK), 0)
        member = (gids == brow)
        soh = member.astype(jnp.float32)
        eoh = (member & (brow != bnext_ref[0])).astype(jnp.float32)
        gmax_s[...] += jnp.dot(eoh, m, preferred_element_type=jnp.float32)
        gsum_s[...] += jnp.dot(soh, h, preferred_element_type=jnp.float32)
        cnt_s[...] += jnp.sum(soh, axis=1, keepdims=True)

        @pl.when(i == _NFB - 1)
        def _():
            gmean = gsum_s[...] / jnp.maximum(cnt_s[...], 1.0)
            cc = (jnp.dot(gmax_s[...], fWa_ref[...], preferred_element_type=jnp.float32)
                  + jnp.dot(gmean, fWb_ref[...], preferred_element_type=jnp.float32)
                  + jnp.dot(mol_ref[...], fWc_ref[...], preferred_element_type=jnp.float32)
                  + fb_ref[...])
            cc = jnp.maximum(cc, 0.0)
            out_ref[...] = (jnp.dot(cc, f2W_ref[...], preferred_element_type=jnp.float32)
                            + f2b_ref[...])

    return pl.pallas_call(
        body,
        grid=(_NFB,),
        in_specs=[
            pl.BlockSpec((2, _FBLK, _F), lambda i: (0, i, 0)),
            pl.BlockSpec((_FBLK, _F), lambda i: (i, 0)),
            pl.BlockSpec((_FBLK, 1), lambda i: (i, 0)),
            pl.BlockSpec((1, _F), lambda i: (0, 0)),
            pl.BlockSpec((_FBLK, 1), lambda i: (i, 0)),
            pl.BlockSpec((1, 1, _FBLK), lambda i: (i, 0, 0)),
            pl.BlockSpec((1, 1, _FBLK), lambda i: (i, 0, 0)),
            pl.BlockSpec((_G, 16), lambda i: (0, 0)),
            pl.BlockSpec((_F, _F), lambda i: (0, 0)),
            pl.BlockSpec((_F, _F), lambda i: (0, 0)),
            pl.BlockSpec((16, _F), lambda i: (0, 0)),
            pl.BlockSpec((1, _F), lambda i: (0, 0)),
            pl.BlockSpec((_F, 1), lambda i: (0, 0)),
            pl.BlockSpec((1, 1), lambda i: (0, 0)),
        ],
        out_specs=pl.BlockSpec((_G, 1), lambda i: (0, 0)),
        out_shape=jax.ShapeDtypeStruct((_G, 1), jnp.float32),
        scratch_shapes=[
            pltpu.VMEM((_G, _F), jnp.float32),
            pltpu.VMEM((_G, _F), jnp.float32),
            pltpu.VMEM((_G, 1), jnp.float32),
            pltpu.VMEM((1, _F), jnp.float32),
            pltpu.VMEM((1, 1), jnp.int32),
        ],
    )(accp, g, dis, b, bcol, brow, bnext, mol, fWa, fWb, fWc, fb, f2W, f2b)


def kernel(x, edge_index, batch, mol_features, W1, b1, W2, b2, W3, b3,
           fc1_W, fc1_b, fc2_W, fc2_b):
    src = edge_index[0].astype(jnp.int32)
    dst = edge_index[1].astype(jnp.int32)
    pad = _E_PAD - _E
    src_p = jnp.concatenate([src, jnp.zeros((pad,), jnp.int32)]).reshape(_TOT_CHUNKS, _CHUNK)
    dst_p = jnp.concatenate([dst, jnp.full((pad,), _N, jnp.int32)]).reshape(_TOT_CHUNKS, _CHUNK)
    bat = batch.astype(jnp.int32)
    bcol = bat.reshape(_N, 1)
    brow = bat.reshape(_NFB, 1, _FBLK)
    bnext = jnp.concatenate([bat[1:], jnp.full((1,), -1, jnp.int32)]).reshape(_NFB, 1, _FBLK)

    u1 = _tc_matmul1(x, W1)
    degp = _sc_degree(dst_p)
    g1, dis = _tc_scale(degp, u1)
    acc1 = _sc_scatter(g1, src_p, dst_p)
    g2 = _tc_mid(acc1, g1, dis, b1.reshape(1, _F), W2)
    acc2 = _sc_scatter(g2, src_p, dst_p)
    g3 = _tc_mid(acc2, g2, dis, b2.reshape(1, _F), W3)
    acc3 = _sc_scatter(g3, src_p, dst_p)
    out = _tc_final(acc3, g3, dis, b3.reshape(1, _F), bcol, brow, bnext,
                    mol_features, fc1_W[:_F], fc1_W[_F:2 * _F], fc1_W[2 * _F:],
                    fc1_b.reshape(1, _F), fc2_W, fc2_b.reshape(1, 1))
    return out[:, 0]


# speedup vs baseline: 2.3854x; 1.0131x over previous
import functools

import jax
import jax.numpy as jnp
from jax import lax
from jax.experimental import pallas as pl
from jax.experimental.pallas import tpu as pltpu
from jax.experimental.pallas import tpu_sc as plsc

_N = 10000
_E = 320000
_G = 200
_F = 64

_NC = 2
_NS = 16
_NW = _NC * _NS
_CHUNK = 128
_TOT_CHUNKS = _E // _CHUNK
_MAXC = 79
_N_ACC = 10112
_RPT = _N_ACC // _NS

@functools.lru_cache(maxsize=None)
def _mesh():
    return plsc.VectorSubcoreMesh(core_axis_name="c", subcore_axis_name="s",
                                  num_cores=_NC, num_subcores=_NS)


def _sc_degree(dst3):
    k = pl.kernel(
        _sc_degree_body,
        mesh=_mesh(),
        out_type=jax.ShapeDtypeStruct((_NC, _N_ACC, 16), jnp.float32),
        scratch_types=[
            pltpu.VMEM((_MAXC, _CHUNK), jnp.int32),
            pltpu.VMEM((_CHUNK, 16), jnp.float32),
            pltpu.VMEM((_RPT, 16), jnp.float32),
            pltpu.VMEM_SHARED((_N_ACC, 16), jnp.float32),
            pltpu.SemaphoreType.DMA,
        ],
        compiler_params=pltpu.CompilerParams(use_tc_tiling_on_sc=False),
    )
    return k(dst3)


def _sc_degree_body(dst_hbm, out_hbm, dsts_v, ones_v, zrows_v, acc_sh, sem):
    cid = lax.axis_index("c")
    sid = lax.axis_index("s")
    one16 = jnp.full((16,), 1.0, jnp.float32)
    zero16 = jnp.zeros((16,), jnp.float32)

    def fill(j, _):
        ones_v[j, :] = one16
        return 0

    lax.fori_loop(0, _CHUNK, fill, 0)

    def zfill(j, _):
        zrows_v[j, :] = zero16
        return 0

    lax.fori_loop(0, _RPT, zfill, 0)

    row_base = sid * _RPT
    pltpu.sync_copy(zrows_v, acc_sh.at[pl.ds(row_base, _RPT)])

    def run(nchunks, start_row):
        if nchunks == 0:
            return
        pltpu.sync_copy(dst_hbm.at[pl.ds(start_row, nchunks)],
                        dsts_v.at[pl.ds(0, nchunks)])

        def body(j, _):
            for b in range(8):
                pltpu.async_copy(ones_v, acc_sh.at[dsts_v.at[j * 8 + b]], sem, add=True)
            for b in range(8):
                pltpu.make_async_copy(ones_v, acc_sh.at[dsts_v.at[j * 8 + b]], sem).wait()
            return 0

        lax.fori_loop(0, nchunks // 8, body, 0)
        rem = nchunks - (nchunks // 8) * 8
        for b in range(rem):
            pltpu.async_copy(ones_v, acc_sh.at[dsts_v.at[nchunks - rem + b]], sem, add=True)
        for b in range(rem):
            pltpu.make_async_copy(ones_v, acc_sh.at[dsts_v.at[nchunks - rem + b]], sem).wait()

    plsc.subcore_barrier()
    _dispatch_chunks(cid, sid, run)

    plsc.subcore_barrier()
    pltpu.sync_copy(acc_sh.at[pl.ds(row_base, _RPT)],
                    out_hbm.at[cid, pl.ds(row_base, _RPT)])


def _dispatch_chunks(cid, sid, run):
    @pl.when(cid == 0)
    def _():
        @pl.when(sid < 4)
        def _():
            run(79, sid * 79)

        @pl.when(sid >= 4)
        def _():
            run(78, 4 * 79 + (sid - 4) * 78)

    @pl.when(cid == 1)
    def _():
        run(78, 4 * 79 + 12 * 78 + sid * 78)


_NBUF = 3


def _sc_scatter(g, src3, dst3):
    k = pl.kernel(
        _sc_scatter_body,
        mesh=_mesh(),
        out_type=jax.ShapeDtypeStruct((_NC, _N_ACC, _F), jnp.float32),
        scratch_types=[
            pltpu.VMEM((_MAXC, _CHUNK), jnp.int32),
            pltpu.VMEM((_MAXC, _CHUNK), jnp.int32),
            pltpu.VMEM((_NBUF, _CHUNK, _F), jnp.float32),
            pltpu.VMEM_SHARED((_N_ACC, _F), jnp.float32),
            pltpu.VMEM_SHARED((_N_ACC, _F), jnp.float32),
            pltpu.SemaphoreType.DMA,
            pltpu.SemaphoreType.DMA,
            pltpu.SemaphoreType.DMA,
            pltpu.SemaphoreType.DMA,
            pltpu.SemaphoreType.DMA,
        ],
        compiler_params=pltpu.CompilerParams(use_tc_tiling_on_sc=False),
    )
    return k(g, src3, dst3)


def _sc_scatter_body(g_hbm, src_hbm, dst_hbm, out_hbm, srcs_v, dsts_v, rows_v,
                     acc_sh, g_sh, gs0, gs1, gs2, gs3, ssem):
    cid = lax.axis_index("c")
    sid = lax.axis_index("s")
    gsems = [gs0, gs1, gs2, gs3]
    zero16 = jnp.zeros((16,), jnp.float32)

    def zfill(j, _):
        for gc in range(_F // 16):
            rows_v[0, j, pl.ds(gc * 16, 16)] = zero16
        return 0

    lax.fori_loop(0, _CHUNK, zfill, 0)

    row_base = sid * _RPT
    for z in range(4):
        pltpu.sync_copy(rows_v.at[0], acc_sh.at[pl.ds(row_base + z * _CHUNK, _CHUNK)])
    pltpu.sync_copy(rows_v.at[0, pl.ds(0, _RPT - 4 * _CHUNK)],
                    acc_sh.at[pl.ds(row_base + 4 * _CHUNK, _RPT - 4 * _CHUNK)])

    @pl.when(sid < _NS - 1)
    def _():
        pltpu.sync_copy(g_hbm.at[pl.ds(row_base, _RPT)],
                        g_sh.at[pl.ds(row_base, _RPT)])

    @pl.when(sid == _NS - 1)
    def _():
        pltpu.sync_copy(g_hbm.at[pl.ds((_NS - 1) * _RPT, _N - (_NS - 1) * _RPT)],
                        g_sh.at[pl.ds((_NS - 1) * _RPT, _N - (_NS - 1) * _RPT)])

    def gstart(c, b):
        pltpu.async_copy(g_sh.at[srcs_v.at[c]], rows_v.at[b], gsems[b])

    def gwait(c, b):
        pltpu.make_async_copy(g_sh.at[srcs_v.at[c]], rows_v.at[b], gsems[b]).wait()

    def sstep(c, b):
        pltpu.async_copy(rows_v.at[b], acc_sh.at[dsts_v.at[c]], ssem, add=True)
        pltpu.make_async_copy(rows_v.at[b], acc_sh.at[dsts_v.at[c]], ssem).wait()

    def run(nchunks, start_row):
        if nchunks == 0:
            return
        pltpu.sync_copy(src_hbm.at[pl.ds(start_row, nchunks)],
                        srcs_v.at[pl.ds(0, nchunks)])
        pltpu.sync_copy(dst_hbm.at[pl.ds(start_row, nchunks)],
                        dsts_v.at[pl.ds(0, nchunks)])
        for b in range(_NBUF):
            gstart(b, b)

        def body(j, _):
            for b in range(_NBUF):
                c = j * _NBUF + b
                gwait(c, b)
                sstep(c, b)
                gstart(c + _NBUF, b)
            return 0

        nfull = (nchunks - _NBUF) // _NBUF
        lax.fori_loop(0, nfull, body, 0)
        for c in range(nfull * _NBUF, nchunks):
            b = c % _NBUF
            gwait(c, b)
            sstep(c, b)
            if c + _NBUF < nchunks:
                gstart(c + _NBUF, b)

    plsc.subcore_barrier()
    _dispatch_chunks(cid, sid, run)

    plsc.subcore_barrier()
    pltpu.sync_copy(acc_sh.at[pl.ds(row_base, _RPT)],
                    out_hbm.at[cid, pl.ds(row_base, _RPT)])


_BLK = 1000


def _tc_matmul1(x, W1):
    def body(x_ref, W1_ref, u_ref):
        u_ref[...] = jnp.dot(x_ref[...], W1_ref[...],
                             preferred_element_type=jnp.float32)

    return pl.pallas_call(
        body,
        grid=(_N // _BLK,),
        in_specs=[
            pl.BlockSpec((_BLK, 128), lambda i: (i, 0)),
            pl.BlockSpec((128, _F), lambda i: (0, 0)),
        ],
        out_specs=pl.BlockSpec((_BLK, _F), lambda i: (i, 0)),
        out_shape=jax.ShapeDtypeStruct((_N, _F), jnp.float32),
    )(x, W1)


def _tc_scale(degp, u1):
    def body(degp_ref, u_ref, g_ref, dis_ref):
        deg = 1.0 + degp_ref[0, :, 0:1] + degp_ref[1, :, 0:1]
        dis = lax.rsqrt(deg)
        dis_ref[...] = dis
        g_ref[...] = u_ref[...] * dis

    return pl.pallas_call(
        body,
        grid=(_N // _BLK,),
        in_specs=[
            pl.BlockSpec((2, _BLK, 16), lambda i: (0, i, 0)),
            pl.BlockSpec((_BLK, _F), lambda i: (i, 0)),
        ],
        out_specs=[
            pl.BlockSpec((_BLK, _F), lambda i: (i, 0)),
            pl.BlockSpec((_BLK, 1), lambda i: (i, 0)),
        ],
        out_shape=[
            jax.ShapeDtypeStruct((_N, _F), jnp.float32),
            jax.ShapeDtypeStruct((_N, 1), jnp.float32),
        ],
    )(degp, u1)


def _tc_mid(accp, g, dis, b, W):
    def body(accp_ref, g_ref, dis_ref, b_ref, W_ref, gn_ref):
        dis = dis_ref[...]
        h = jnp.maximum(dis * (accp_ref[0] + accp_ref[1] + g_ref[...]) + b_ref[...], 0.0)
        gn_ref[...] = jnp.dot(h, W_ref[...], preferred_element_type=jnp.float32) * dis

    return pl.pallas_call(
        body,
        grid=(_N // _BLK,),
        in_specs=[
            pl.BlockSpec((2, _BLK, _F), lambda i: (0, i, 0)),
            pl.BlockSpec((_BLK, _F), lambda i: (i, 0)),
            pl.BlockSpec((_BLK, 1), lambda i: (i, 0)),
            pl.BlockSpec((1, _F), lambda i: (0, 0)),
            pl.BlockSpec((_F, _F), lambda i: (0, 0)),
        ],
        out_specs=pl.BlockSpec((_BLK, _F), lambda i: (i, 0)),
        out_shape=jax.ShapeDtypeStruct((_N, _F), jnp.float32),
    )(accp, g, dis, b, W)


_FBLK = 1000
_NFB = _N // _FBLK


def _tc_final(accp, g, dis, b, bcol, brow, bnext, mol, fWa, fWb, fWc, fb, f2W, f2b):
    def body(accp_ref, g_ref, dis_ref, b_ref, bcol_ref, brow_ref, bnext_ref,
             mol_ref, fWa_ref, fWb_ref, fWc_ref, fb_ref, f2W_ref, f2b_ref,
             out_ref, gmax_s, gsum_s, cnt_s, cm_s, cb_s):
        i = pl.program_id(0)

        @pl.when(i == 0)
        def _():
            gmax_s[...] = jnp.zeros((_G, _F), jnp.float32)
            gsum_s[...] = jnp.zeros((_G, _F), jnp.float32)
            cnt_s[...] = jnp.zeros((_G, 1), jnp.float32)
            cm_s[...] = jnp.zeros((1, _F), jnp.float32)
            cb_s[...] = jnp.full((1, 1), -1, jnp.int32)

        h = jnp.maximum(
            dis_ref[...] * (accp_ref[0] + accp_ref[1] + g_ref[...]) + b_ref[...], 0.0)
        bcol = bcol_ref[...]
        m = h
        for k in range(10):
            sh = 1 << k
            m_sh = jnp.concatenate(
                [jnp.zeros((sh, _F), jnp.float32), m[:_FBLK - sh]], axis=0)
            b_sh = jnp.concatenate(
                [jnp.full((sh, 1), -1, jnp.int32), bcol[:_FBLK - sh]], axis=0)
            m = jnp.where(bcol == b_sh, jnp.maximum(m, m_sh), m)
        m = jnp.where(bcol == cb_s[...], jnp.maximum(m, cm_s[...]), m)
        cm_s[...] = m[_FBLK - 1:_FBLK]
        cb_s[...] = bcol[_FBLK - 1:_FBLK]

        brow = brow_ref[0]
        gids = lax.broadcasted_iota(jnp.int32, (_G, _FBLK), 0)
        member = (gids == brow)
        soh = member.astype(jnp.float32)
        eoh = (member & (brow != bnext_ref[0])).astype(jnp.float32)
        gmax_s[...] += jnp.dot(eoh, m, preferred_element_type=jnp.float32)
        gsum_s[...] += jnp.dot(soh, h, preferred_element_type=jnp.float32)
        cnt_s[...] += jnp.sum(soh, axis=1, keepdims=True)

        @pl.when(i == _NFB - 1)
        def _():
            gmean = gsum_s[...] / jnp.maximum(cnt_s[...], 1.0)
            cc = (jnp.dot(gmax_s[...], fWa_ref[...], preferred_element_type=jnp.float32)
                  + jnp.dot(gmean, fWb_ref[...], preferred_element_type=jnp.float32)
                  + jnp.dot(mol_ref[...], fWc_ref[...], preferred_element_type=jnp.float32)
                  + fb_ref[...])
            cc = jnp.maximum(cc, 0.0)
            out_ref[...] = (jnp.dot(cc, f2W_ref[...], preferred_element_type=jnp.float32)
                            + f2b_ref[...])

    return pl.pallas_call(
        body,
        grid=(_NFB,),
        in_specs=[
            pl.BlockSpec((2, _FBLK, _F), lambda i: (0, i, 0)),
            pl.BlockSpec((_FBLK, _F), lambda i: (i, 0)),
            pl.BlockSpec((_FBLK, 1), lambda i: (i, 0)),
            pl.BlockSpec((1, _F), lambda i: (0, 0)),
            pl.BlockSpec((_FBLK, 1), lambda i: (i, 0)),
            pl.BlockSpec((1, 1, _FBLK), lambda i: (i, 0, 0)),
            pl.BlockSpec((1, 1, _FBLK), lambda i: (i, 0, 0)),
            pl.BlockSpec((_G, 16), lambda i: (0, 0)),
            pl.BlockSpec((_F, _F), lambda i: (0, 0)),
            pl.BlockSpec((_F, _F), lambda i: (0, 0)),
            pl.BlockSpec((16, _F), lambda i: (0, 0)),
            pl.BlockSpec((1, _F), lambda i: (0, 0)),
            pl.BlockSpec((_F, 1), lambda i: (0, 0)),
            pl.BlockSpec((1, 1), lambda i: (0, 0)),
        ],
        out_specs=pl.BlockSpec((_G, 1), lambda i: (0, 0)),
        out_shape=jax.ShapeDtypeStruct((_G, 1), jnp.float32),
        scratch_shapes=[
            pltpu.VMEM((_G, _F), jnp.float32),
            pltpu.VMEM((_G, _F), jnp.float32),
            pltpu.VMEM((_G, 1), jnp.float32),
            pltpu.VMEM((1, _F), jnp.float32),
            pltpu.VMEM((1, 1), jnp.int32),
        ],
    )(accp, g, dis, b, bcol, brow, bnext, mol, fWa, fWb, fWc, fb, f2W, f2b)


def kernel(x, edge_index, batch, mol_features, W1, b1, W2, b2, W3, b3,
           fc1_W, fc1_b, fc2_W, fc2_b):
    src_p = edge_index[0].astype(jnp.int32).reshape(_TOT_CHUNKS, _CHUNK)
    dst_p = edge_index[1].astype(jnp.int32).reshape(_TOT_CHUNKS, _CHUNK)
    bat = batch.astype(jnp.int32)
    bcol = bat.reshape(_N, 1)
    brow = bat.reshape(_NFB, 1, _FBLK)
    bnext = jnp.concatenate([bat[1:], jnp.full((1,), -1, jnp.int32)]).reshape(_NFB, 1, _FBLK)

    u1 = _tc_matmul1(x, W1)
    degp = _sc_degree(dst_p)
    g1, dis = _tc_scale(degp, u1)
    acc1 = _sc_scatter(g1, src_p, dst_p)
    g2 = _tc_mid(acc1, g1, dis, b1.reshape(1, _F), W2)
    acc2 = _sc_scatter(g2, src_p, dst_p)
    g3 = _tc_mid(acc2, g2, dis, b2.reshape(1, _F), W3)
    acc3 = _sc_scatter(g3, src_p, dst_p)
    out = _tc_final(acc3, g3, dis, b3.reshape(1, _F), bcol, brow, bnext,
                    mol_features, fc1_W[:_F], fc1_W[_F:2 * _F], fc1_W[2 * _F:],
                    fc1_b.reshape(1, _F), fc2_W, fc2_b.reshape(1, 1))
    return out[:, 0]
